# Initial kernel scaffold; baseline (speedup 1.0000x reference)
import jax
import jax.numpy as jnp
from jax.experimental import pallas as pl


def kernel(x, edge_index, batch, gat1_Wl, gat1_Wr, gat1_att, gat1_b, proj1_W, proj1_b, gat2_Wl, gat2_Wr, gat2_att, gat2_b, gat3_Wl, gat3_Wr, gat3_att, gat3_b, gat4_Wl, gat4_Wr, gat4_att, gat4_b, gat5_Wl, gat5_Wr, gat5_att, gat5_b, bn1_g, bn1_b, bn2_g, bn2_b, bn3_g, bn3_b, bn4_g, bn4_b, bn5_g, bn5_b, lin_W, lin_b):
    raise NotImplementedError("write your pallas kernel here")


# speedup vs baseline: 9.0279x; 9.0279x over previous
import functools

import jax
import jax.numpy as jnp
from jax import lax
from jax.experimental import pallas as pl
from jax.experimental.pallas import tpu as pltpu
from jax.experimental.pallas import tpu_sc as plsc

N = 10000
E = 160000
E2 = E + N
DIN = 128
HID = 128
HEADS = 4
NCLS = 7
NG = 64

NP = 10112
GW = 128
NW = 32
NGRP = 42
NGS = 48
CHS = NGS * GW
E2P = NW * CHS
CNT_LO = E2 // NW
CNT_HI = CNT_LO + 1
ROWS_T = NP // 16
NPD = 80

_f32 = jnp.float32
_i32 = jnp.int32


def _edge_map():
    import numpy as np
    m = np.full((E2P,), E2, np.int32)
    pos = 0
    for w in range(NW):
        c = CNT_HI if w < 16 else CNT_LO
        m[w * CHS:w * CHS + c] = np.arange(pos, pos + c, dtype=np.int32)
        pos += c
    assert pos == E2
    return m


_EDGE_MAP = _edge_map()


@functools.lru_cache(maxsize=None)
def _sc_scores(H):
    mesh = plsc.VectorSubcoreMesh(core_axis_name="c", subcore_axis_name="s")

    def body(xlh, xrh, src2d, dst2d, att_h, ex_h, denp,
             src_v, dst_v, idxs0, idxs1, idxd0, idxd1, xl0, xl1, xr0, xr1,
             ex_buf, stage_den, idx_den, att_v, acc_den,
             sl0, sl1, sr0, sr1):
        c = lax.axis_index("c")
        s = lax.axis_index("s")
        wid = s * 2 + c
        cnt = jnp.where(wid < 16, CNT_HI, CNT_LO)
        pltpu.sync_copy(src2d.at[pl.ds(wid * NGS, NGS)], src_v)
        pltpu.sync_copy(dst2d.at[pl.ds(wid * NGS, NGS)], dst_v)
        pltpu.sync_copy(att_h, att_v)
        iota = lax.iota(_i32, 16)
        zv = jnp.zeros((16,), _f32)
        slots = ((idxs0, idxd0, xl0, xr0, sl0, sr0),
                 (idxs1, idxd1, xl1, xr1, sl1, sr1))

        def zrow(i, _):
            for v in range(HID // 16):
                stage_den[i, pl.ds(v * 16, 16)] = zv
            return 0

        def issue(g, slot, h):
            idxs, idxd, xl_b, xr_b, s_l, s_r = slot
            for j in range(8):
                sv = src_v[g, pl.ds(j * 16, 16)]
                idxs[pl.ds(j * 16, 16)] = sv * H + h
                dv = dst_v[g, pl.ds(j * 16, 16)]
                idxd[pl.ds(j * 16, 16)] = dv * H + h
            pltpu.async_copy(xlh.at[idxs], xl_b, s_l)
            pltpu.async_copy(xrh.at[idxd], xr_b, s_r)

        def process(g, slot, h, att_k):
            idxs, idxd, xl_b, xr_b, s_l, s_r = slot
            pltpu.make_async_copy(xlh.at[idxs], xl_b, s_l).wait()
            pltpu.make_async_copy(xrh.at[idxd], xr_b, s_r).wait()
            for sg in range(8):

                def edge(i, sv, sg=sg):
                    e = sg * 16 + i
                    acc = jnp.zeros((16,), _f32)
                    for k in range(HID // 16):
                        m = (xl_b[e, pl.ds(k * 16, 16)]
                             + xr_b[e, pl.ds(k * 16, 16)])
                        m = jnp.maximum(m, m * 0.2)
                        acc = acc + m * att_k[k]
                    for b in (1, 2, 4, 8):
                        perm = jnp.bitwise_xor(iota, b)
                        acc = acc + acc.at[perm].get(
                            mode="promise_in_bounds")
                    return jnp.where(iota == i, acc, sv)

                score = lax.fori_loop(0, 16, edge, jnp.zeros((16,), _f32), unroll=2)
                le = iota + (g * GW + sg * 16)
                exv = jnp.where(le < cnt, jnp.exp(score), 0.0)
                ex_buf[pl.ds(sg * 16, 16)] = exv
            pltpu.sync_copy(
                ex_buf, ex_h.at[pl.ds(h * E2P + wid * CHS + g * GW, GW)])
            @pl.when(g < NGRP - 2)
            def _():
                issue(g + 2, slot, h)
            for j in range(8):
                dv = dst_v[g, pl.ds(j * 16, 16)]
                idx_den[pl.ds(j * 16, 16)] = jnp.right_shift(dv, 7)
            for sg in range(8):
                exv = ex_buf[pl.ds(sg * 16, 16)]
                dmod = jnp.bitwise_and(dst_v[g, pl.ds(sg * 16, 16)], 127)

                def row(i, _, exv=exv, dmod=dmod, sg=sg):
                    e = sg * 16 + i
                    lane = jnp.full((16,), i, _i32)
                    sc = exv.at[lane].get(mode="promise_in_bounds")
                    dm = dmod.at[lane].get(mode="promise_in_bounds")
                    for v in range(HID // 16):
                        stage_den[e, pl.ds(v * 16, 16)] = jnp.where(
                            iota + (v * 16) == dm, sc, 0.0)
                    return 0

                lax.fori_loop(0, 16, row, 0, unroll=2)
            pltpu.sync_copy(stage_den, acc_den.at[idx_den], add=True)

        def head(h, _):
            att_k = [att_v[h, pl.ds(k * 16, 16)] for k in range(HID // 16)]

            @pl.when(s == 0)
            def _():
                lax.fori_loop(0, NPD, zrow, 0)
                pltpu.sync_copy(stage_den.at[pl.ds(0, NPD)], acc_den)

            plsc.subcore_barrier()
            issue(0, slots[0], h)
            issue(1, slots[1], h)

            def pair(i, _, h=h, att_k=att_k):
                process(i * 2, slots[0], h, att_k)
                process(i * 2 + 1, slots[1], h, att_k)
                return 0

            lax.fori_loop(0, NGRP // 2, pair, 0)
            plsc.subcore_barrier()

            @pl.when(s == 0)
            def _():
                pltpu.sync_copy(acc_den, denp.at[c, h])

            plsc.subcore_barrier()
            return 0

        lax.fori_loop(0, H, head, 0)

    return pl.kernel(
        body,
        out_type=(jax.ShapeDtypeStruct((H * E2P,), _f32),
                  jax.ShapeDtypeStruct((2, H, NPD, HID), _f32)),
        mesh=mesh,
        scratch_types=[
            pltpu.VMEM((NGS, GW), _i32),
            pltpu.VMEM((NGS, GW), _i32),
            pltpu.VMEM((GW,), _i32),
            pltpu.VMEM((GW,), _i32),
            pltpu.VMEM((GW,), _i32),
            pltpu.VMEM((GW,), _i32),
            pltpu.VMEM((GW, HID), _f32),
            pltpu.VMEM((GW, HID), _f32),
            pltpu.VMEM((GW, HID), _f32),
            pltpu.VMEM((GW, HID), _f32),
            pltpu.VMEM((GW,), _f32),
            pltpu.VMEM((GW, HID), _f32),
            pltpu.VMEM((GW,), _i32),
            pltpu.VMEM((H, HID), _f32),
            pltpu.VMEM_SHARED((NPD, HID), _f32),
            pltpu.SemaphoreType.DMA,
            pltpu.SemaphoreType.DMA,
            pltpu.SemaphoreType.DMA,
            pltpu.SemaphoreType.DMA,
        ],
    )


@functools.lru_cache(maxsize=None)
def _sc_aggregate(H):
    mesh = plsc.VectorSubcoreMesh(core_axis_name="c", subcore_axis_name="s")

    def body(xlh, src2d, dst2d, ex_h, outp,
             src_v, dst_v, idxs0, idxs1, ex0, ex1, xl0, xl1,
             acc, sl0, sl1, se0, se1):
        c = lax.axis_index("c")
        s = lax.axis_index("s")
        wid = s * 2 + c
        zv = jnp.zeros((16,), _f32)
        pltpu.sync_copy(src2d.at[pl.ds(wid * NGS, NGS)], src_v)
        pltpu.sync_copy(dst2d.at[pl.ds(wid * NGS, NGS)], dst_v)
        slots = ((idxs0, ex0, xl0, sl0, se0),
                 (idxs1, ex1, xl1, sl1, se1))

        def zrow(i, _):
            for v in range(HID // 16):
                xl0[i, pl.ds(v * 16, 16)] = zv
            return 0

        def issue(g, slot, h):
            idxs, ex_b, xl_b, s_l, s_e = slot
            for j in range(8):
                sv = src_v[g, pl.ds(j * 16, 16)]
                idxs[pl.ds(j * 16, 16)] = sv * H + h
            pltpu.async_copy(xlh.at[idxs], xl_b, s_l)
            pltpu.async_copy(
                ex_h.at[pl.ds(h * E2P + wid * CHS + g * GW, GW)], ex_b, s_e)

        def process(g, slot, h):
            idxs, ex_b, xl_b, s_l, s_e = slot
            pltpu.make_async_copy(xlh.at[idxs], xl_b, s_l).wait()
            pltpu.make_async_copy(
                ex_h.at[pl.ds(h * E2P + wid * CHS + g * GW, GW)],
                ex_b, s_e).wait()
            for sg in range(8):
                exv = ex_b[pl.ds(sg * 16, 16)]

                def row(i, _, exv=exv, sg=sg):
                    e = sg * 16 + i
                    lane = jnp.full((16,), i, _i32)
                    sc = exv.at[lane].get(mode="promise_in_bounds")
                    for v in range(HID // 16):
                        xl_b[e, pl.ds(v * 16, 16)] = (
                            xl_b[e, pl.ds(v * 16, 16)] * sc)
                    return 0

                lax.fori_loop(0, 16, row, 0, unroll=2)
            pltpu.sync_copy(xl_b, acc.at[dst_v.at[g]], add=True)

            @pl.when(g < NGRP - 2)
            def _():
                issue(g + 2, slot, h)

        def head(h, _):
            lax.fori_loop(0, GW, zrow, 0)
            for zoff, zlen in ((0, 128), (128, 128), (256, 128),
                               (384, 128), (512, 120)):
                pltpu.sync_copy(
                    xl0.at[pl.ds(0, zlen)],
                    acc.at[pl.ds(s * ROWS_T + zoff, zlen)])
            plsc.subcore_barrier()
            issue(0, slots[0], h)
            issue(1, slots[1], h)

            def pair(i, _, h=h):
                process(i * 2, slots[0], h)
                process(i * 2 + 1, slots[1], h)
                return 0

            lax.fori_loop(0, NGRP // 2, pair, 0)
            plsc.subcore_barrier()
            pltpu.sync_copy(acc.at[pl.ds(s * ROWS_T, ROWS_T)],
                            outp.at[c, h, pl.ds(s * ROWS_T, ROWS_T)])
            plsc.subcore_barrier()
            return 0

        lax.fori_loop(0, H, head, 0)

    return pl.kernel(
        body,
        out_type=(jax.ShapeDtypeStruct((2, H, NP, HID), _f32),),
        mesh=mesh,
        scratch_types=[
            pltpu.VMEM((NGS, GW), _i32),
            pltpu.VMEM((NGS, GW), _i32),
            pltpu.VMEM((GW,), _i32),
            pltpu.VMEM((GW,), _i32),
            pltpu.VMEM((GW,), _f32),
            pltpu.VMEM((GW,), _f32),
            pltpu.VMEM((GW, HID), _f32),
            pltpu.VMEM((GW, HID), _f32),
            pltpu.VMEM_SHARED((NP, HID), _f32),
            pltpu.SemaphoreType.DMA,
            pltpu.SemaphoreType.DMA,
            pltpu.SemaphoreType.DMA,
            pltpu.SemaphoreType.DMA,
        ],
    )


def _mm2(x, w1, w2):
    npad, k = x.shape
    f = w1.shape[1]
    br = 128
    grid = (npad // br,)

    def body(x_ref, w1_ref, w2_ref, o1_ref, o2_ref):
        xb = x_ref[...]
        o1_ref[...] = jnp.dot(xb, w1_ref[...], preferred_element_type=_f32)
        o2_ref[...] = jnp.dot(xb, w2_ref[...], preferred_element_type=_f32)

    return pl.pallas_call(
        body,
        grid=grid,
        in_specs=[
            pl.BlockSpec((br, k), lambda i: (i, 0)),
            pl.BlockSpec((k, f), lambda i: (0, 0)),
            pl.BlockSpec((k, f), lambda i: (0, 0)),
        ],
        out_specs=[pl.BlockSpec((br, f), lambda i: (i, 0))] * 2,
        out_shape=[jax.ShapeDtypeStruct((npad, f), _f32)] * 2,
    )(x, w1, w2)


def _combine_stats(outp, denp, H):
    f = H * HID
    br = 128
    grid = (NP // br,)

    def body(op_ref, dp_ref, y_ref, st_ref, acc_ref):
        g = pl.program_id(0)
        num = op_ref[0] + op_ref[1]
        den = dp_ref[0] + dp_ref[1]
        yb = num.transpose(1, 0, 2) / (den.T[:, :, None] + 1e-16)
        yb = yb.reshape(br, f)
        y_ref[...] = yb

        @pl.when(g == 0)
        def _():
            acc_ref[...] = jnp.zeros_like(acc_ref)

        acc_ref[0, :] += jnp.sum(yb, axis=0)
        acc_ref[1, :] += jnp.sum(yb * yb, axis=0)

        @pl.when(g == NP // br - 1)
        def _():
            st_ref[...] = acc_ref[...]

    return pl.pallas_call(
        body,
        grid=grid,
        in_specs=[
            pl.BlockSpec((2, H, br, HID), lambda i: (0, 0, i, 0)),
            pl.BlockSpec((2, H, br), lambda i: (0, 0, i)),
        ],
        out_specs=[
            pl.BlockSpec((br, f), lambda i: (i, 0)),
            pl.BlockSpec((8, f), lambda i: (0, 0)),
        ],
        out_shape=[
            jax.ShapeDtypeStruct((NP, f), _f32),
            jax.ShapeDtypeStruct((8, f), _f32),
        ],
        scratch_shapes=[pltpu.VMEM((8, f), _f32)],
    )(outp, denp)


def _bn_block(y, st, g, b):
    mu = st[0, :] * (1.0 / N)
    var = st[1, :] * (1.0 / N) - mu * mu
    inv = lax.rsqrt(var + 1e-5)
    return (y - mu[None, :]) * (inv * g)[None, :] + b[None, :]


def _norm_proj_mm(y, st, bn_g, bn_b, pw, pb, wl, wr):
    f = y.shape[1]
    br = 128
    grid = (NP // br,)

    def body(y_ref, st_ref, g_ref, b_ref, pw_ref, pb_ref, wl_ref, wr_ref,
             xp_ref, xl_ref, xr_ref):
        x1 = jnp.maximum(
            _bn_block(y_ref[...], st_ref[...], g_ref[0], b_ref[0]), 0.0)
        xp = jnp.dot(x1, pw_ref[...], preferred_element_type=_f32) + pb_ref[0]
        xp_ref[...] = xp
        xl_ref[...] = jnp.dot(xp, wl_ref[...], preferred_element_type=_f32)
        xr_ref[...] = jnp.dot(xp, wr_ref[...], preferred_element_type=_f32)

    return pl.pallas_call(
        body,
        grid=grid,
        in_specs=[
            pl.BlockSpec((br, f), lambda i: (i, 0)),
            pl.BlockSpec((8, f), lambda i: (0, 0)),
            pl.BlockSpec((1, f), lambda i: (0, 0)),
            pl.BlockSpec((1, f), lambda i: (0, 0)),
            pl.BlockSpec((f, HID), lambda i: (0, 0)),
            pl.BlockSpec((1, HID), lambda i: (0, 0)),
            pl.BlockSpec((HID, HID), lambda i: (0, 0)),
            pl.BlockSpec((HID, HID), lambda i: (0, 0)),
        ],
        out_specs=[pl.BlockSpec((br, HID), lambda i: (i, 0))] * 3,
        out_shape=[jax.ShapeDtypeStruct((NP, HID), _f32)] * 3,
    )(y, st, bn_g, bn_b, pw, pb, wl, wr)


def _norm_res_mm(y, st, bn_g, bn_b, res, wl, wr):
    br = 128
    grid = (NP // br,)

    def body(y_ref, st_ref, g_ref, b_ref, r_ref, wl_ref, wr_ref,
             xt_ref, xl_ref, xr_ref):
        xt = jnp.maximum(
            _bn_block(y_ref[...], st_ref[...], g_ref[0], b_ref[0]), 0.0)
        xt = xt + r_ref[...]
        xt_ref[...] = xt
        xl_ref[...] = jnp.dot(xt, wl_ref[...], preferred_element_type=_f32)
        xr_ref[...] = jnp.dot(xt, wr_ref[...], preferred_element_type=_f32)

    return pl.pallas_call(
        body,
        grid=grid,
        in_specs=[
            pl.BlockSpec((br, HID), lambda i: (i, 0)),
            pl.BlockSpec((8, HID), lambda i: (0, 0)),
            pl.BlockSpec((1, HID), lambda i: (0, 0)),
            pl.BlockSpec((1, HID), lambda i: (0, 0)),
            pl.BlockSpec((br, HID), lambda i: (i, 0)),
            pl.BlockSpec((HID, HID), lambda i: (0, 0)),
            pl.BlockSpec((HID, HID), lambda i: (0, 0)),
        ],
        out_specs=[pl.BlockSpec((br, HID), lambda i: (i, 0))] * 3,
        out_shape=[jax.ShapeDtypeStruct((NP, HID), _f32)] * 3,
    )(y, st, bn_g, bn_b, res, wl, wr)


def _norm_res_pool(y, st, bn_g, bn_b, res, batch3d, lw, lb):
    br = 128
    grid = (NP // br,)

    def body(y_ref, st_ref, g_ref, b_ref, r_ref, bt_ref, lw_ref, lb_ref,
             o_ref, acc_ref):
        g = pl.program_id(0)
        x5 = jnp.maximum(
            _bn_block(y_ref[...], st_ref[...], g_ref[0], b_ref[0]), 0.0)
        x5 = x5 + r_ref[...]
        bt = bt_ref[0]
        oh = jnp.where(
            bt.reshape(br, 1) == lax.broadcasted_iota(_i32, (br, NG), 1),
            1.0, 0.0)

        @pl.when(g == 0)
        def _():
            acc_ref[...] = jnp.zeros_like(acc_ref)

        acc_ref[...] += lax.dot_general(oh, x5, (((0,), (0,)), ((), ())),
                                        preferred_element_type=_f32)

        @pl.when(g == NP // br - 1)
        def _():
            o_ref[...] = jnp.dot(acc_ref[...], lw_ref[...],
                                 preferred_element_type=_f32) + lb_ref[0]

    return pl.pallas_call(
        body,
        grid=grid,
        in_specs=[
            pl.BlockSpec((br, HID), lambda i: (i, 0)),
            pl.BlockSpec((8, HID), lambda i: (0, 0)),
            pl.BlockSpec((1, HID), lambda i: (0, 0)),
            pl.BlockSpec((1, HID), lambda i: (0, 0)),
            pl.BlockSpec((br, HID), lambda i: (i, 0)),
            pl.BlockSpec((1, 1, br), lambda i: (i, 0, 0)),
            pl.BlockSpec((HID, 128), lambda i: (0, 0)),
            pl.BlockSpec((1, 128), lambda i: (0, 0)),
        ],
        out_specs=[pl.BlockSpec((NG, 128), lambda i: (0, 0))],
        out_shape=[jax.ShapeDtypeStruct((NG, 128), _f32)],
        scratch_shapes=[pltpu.VMEM((NG, HID), _f32)],
    )(y, st, bn_g, bn_b, res, batch3d, lw, lb)


def _gat_layer(xl, xr, src2d, dst2d, att, H):
    xlh = xl.reshape(NP * H, HID)
    xrh = xr.reshape(NP * H, HID)
    ex, denp = _sc_scores(H)(xlh, xrh, src2d, dst2d, att)
    (outp,) = _sc_aggregate(H)(xlh, src2d, dst2d, ex)
    denp = denp.reshape(2, H, NPD * HID)[:, :, :NP]
    return _combine_stats(outp, denp, H)


def kernel(x, edge_index, batch, gat1_Wl, gat1_Wr, gat1_att, gat1_b,
           proj1_W, proj1_b, gat2_Wl, gat2_Wr, gat2_att, gat2_b,
           gat3_Wl, gat3_Wr, gat3_att, gat3_b, gat4_Wl, gat4_Wr, gat4_att,
           gat4_b, gat5_Wl, gat5_Wr, gat5_att, gat5_b, bn1_g, bn1_b,
           bn2_g, bn2_b, bn3_g, bn3_b, bn4_g, bn4_b, bn5_g, bn5_b,
           lin_W, lin_b):
    loop = jnp.arange(N, dtype=jnp.int32)
    src = jnp.concatenate([edge_index[0], loop, jnp.zeros((1,), jnp.int32)])
    dst = jnp.concatenate([edge_index[1], loop, jnp.zeros((1,), jnp.int32)])
    src_p = src[_EDGE_MAP]
    dst_p = dst[_EDGE_MAP]
    src2d = src_p.reshape(E2P // GW, GW)
    dst2d = dst_p.reshape(E2P // GW, GW)
    xp0 = jnp.pad(x, ((0, NP - N), (0, 0)))
    batch3d = jnp.pad(batch, (0, NP - N), constant_values=NG).reshape(
        NP // GW, 1, GW)
    row = lambda v: v.reshape(1, -1)
    lwp = jnp.pad(lin_W, ((0, 0), (0, 128 - NCLS)))
    lbp = jnp.pad(lin_b, (0, 128 - NCLS)).reshape(1, 128)

    xl1, xr1 = _mm2(xp0, gat1_Wl, gat1_Wr)
    y1, st1 = _gat_layer(xl1, xr1, src2d, dst2d, gat1_att, HEADS)
    x1p, xl2, xr2 = _norm_proj_mm(y1, st1, row(bn1_g), row(bn1_b),
                                  proj1_W, row(proj1_b), gat2_Wl, gat2_Wr)

    y2, st2 = _gat_layer(xl2, xr2, src2d, dst2d, gat2_att, 1)
    x2, xl3, xr3 = _norm_res_mm(y2, st2, row(bn2_g), row(bn2_b), x1p,
                                gat3_Wl, gat3_Wr)
    y3, st3 = _gat_layer(xl3, xr3, src2d, dst2d, gat3_att, 1)
    x3, xl4, xr4 = _norm_res_mm(y3, st3, row(bn3_g), row(bn3_b), x2,
                                gat4_Wl, gat4_Wr)
    y4, st4 = _gat_layer(xl4, xr4, src2d, dst2d, gat4_att, 1)
    x4, xl5, xr5 = _norm_res_mm(y4, st4, row(bn4_g), row(bn4_b), x3,
                                gat5_Wl, gat5_Wr)

    y5, st5 = _gat_layer(xl5, xr5, src2d, dst2d, gat5_att, 1)
    (outp,) = _norm_res_pool(y5, st5, row(bn5_g), row(bn5_b), x4,
                             batch3d, lwp, lbp)
    return outp[:, :NCLS]


# speedup vs baseline: 9.0903x; 1.0069x over previous
import functools

import jax
import jax.numpy as jnp
from jax import lax
from jax.experimental import pallas as pl
from jax.experimental.pallas import tpu as pltpu
from jax.experimental.pallas import tpu_sc as plsc

N = 10000
E = 160000
E2 = E + N
DIN = 128
HID = 128
HEADS = 4
NCLS = 7
NG = 64

NP = 10112
GW = 128
NW = 32
NGRP = 42
NGS = 48
CHS = NGS * GW
E2P = NW * CHS
CNT_LO = E2 // NW
CNT_HI = CNT_LO + 1
ROWS_T = NP // 16
NPD = 80

_f32 = jnp.float32
_i32 = jnp.int32


def _edge_map():
    import numpy as np
    m = np.full((E2P,), E2, np.int32)
    pos = 0
    for w in range(NW):
        c = CNT_HI if w < 16 else CNT_LO
        m[w * CHS:w * CHS + c] = np.arange(pos, pos + c, dtype=np.int32)
        pos += c
    assert pos == E2
    return m


_EDGE_MAP = _edge_map()


@functools.lru_cache(maxsize=None)
def _sc_scores(H):
    mesh = plsc.VectorSubcoreMesh(core_axis_name="c", subcore_axis_name="s")

    def body(xlh, xrh, src2d, dst2d, att_h, ex_h, denp,
             src_v, dst_v, idxs0, idxs1, idxd0, idxd1, xl0, xl1, xr0, xr1,
             ex_buf, stage_den, idx_den, att_v, acc_den,
             sl0, sl1, sr0, sr1):
        c = lax.axis_index("c")
        s = lax.axis_index("s")
        wid = s * 2 + c
        cnt = jnp.where(wid < 16, CNT_HI, CNT_LO)
        pltpu.sync_copy(src2d.at[pl.ds(wid * NGS, NGS)], src_v)
        pltpu.sync_copy(dst2d.at[pl.ds(wid * NGS, NGS)], dst_v)
        pltpu.sync_copy(att_h, att_v)
        iota = lax.iota(_i32, 16)
        zv = jnp.zeros((16,), _f32)
        slots = ((idxs0, idxd0, xl0, xr0, sl0, sr0),
                 (idxs1, idxd1, xl1, xr1, sl1, sr1))

        def zrow(i, _):
            for v in range(HID // 16):
                stage_den[i, pl.ds(v * 16, 16)] = zv
            return 0

        def issue(g, slot, h):
            idxs, idxd, xl_b, xr_b, s_l, s_r = slot
            for j in range(8):
                sv = src_v[g, pl.ds(j * 16, 16)]
                idxs[pl.ds(j * 16, 16)] = sv * H + h
                dv = dst_v[g, pl.ds(j * 16, 16)]
                idxd[pl.ds(j * 16, 16)] = dv * H + h
            pltpu.async_copy(xlh.at[idxs], xl_b, s_l)
            pltpu.async_copy(xrh.at[idxd], xr_b, s_r)

        def process(g, slot, h, att_k):
            idxs, idxd, xl_b, xr_b, s_l, s_r = slot
            pltpu.make_async_copy(xlh.at[idxs], xl_b, s_l).wait()
            pltpu.make_async_copy(xrh.at[idxd], xr_b, s_r).wait()
            for sg in range(8):

                def edge(i, sv, sg=sg):
                    res = sv
                    accs = []
                    for d in (0, 1):
                        e = sg * 16 + i * 2 + d
                        a0 = jnp.zeros((16,), _f32)
                        a1 = jnp.zeros((16,), _f32)
                        for k in range(0, HID // 16, 2):
                            m0 = (xl_b[e, pl.ds(k * 16, 16)]
                                  + xr_b[e, pl.ds(k * 16, 16)])
                            m0 = jnp.maximum(m0, m0 * 0.2)
                            a0 = a0 + m0 * att_k[k]
                            m1 = (xl_b[e, pl.ds(k * 16 + 16, 16)]
                                  + xr_b[e, pl.ds(k * 16 + 16, 16)])
                            m1 = jnp.maximum(m1, m1 * 0.2)
                            a1 = a1 + m1 * att_k[k + 1]
                        accs.append(a0 + a1)
                    for d in (0, 1):
                        acc = accs[d]
                        for b in (1, 2, 4, 8):
                            perm = jnp.bitwise_xor(iota, b)
                            acc = acc + acc.at[perm].get(
                                mode="promise_in_bounds")
                        res = jnp.where(iota == i * 2 + d, acc, res)
                    return res

                score = lax.fori_loop(0, 8, edge, jnp.zeros((16,), _f32))
                le = iota + (g * GW + sg * 16)
                exv = jnp.where(le < cnt, jnp.exp(score), 0.0)
                ex_buf[pl.ds(sg * 16, 16)] = exv
            pltpu.sync_copy(
                ex_buf, ex_h.at[pl.ds(h * E2P + wid * CHS + g * GW, GW)])
            @pl.when(g < NGRP - 2)
            def _():
                issue(g + 2, slot, h)
            for j in range(8):
                dv = dst_v[g, pl.ds(j * 16, 16)]
                idx_den[pl.ds(j * 16, 16)] = jnp.right_shift(dv, 7)
            for sg in range(8):
                exv = ex_buf[pl.ds(sg * 16, 16)]
                dmod = jnp.bitwise_and(dst_v[g, pl.ds(sg * 16, 16)], 127)

                def row(i, _, exv=exv, dmod=dmod, sg=sg):
                    e = sg * 16 + i
                    lane = jnp.full((16,), i, _i32)
                    sc = exv.at[lane].get(mode="promise_in_bounds")
                    dm = dmod.at[lane].get(mode="promise_in_bounds")
                    for v in range(HID // 16):
                        stage_den[e, pl.ds(v * 16, 16)] = jnp.where(
                            iota + (v * 16) == dm, sc, 0.0)
                    return 0

                lax.fori_loop(0, 16, row, 0)
            pltpu.sync_copy(stage_den, acc_den.at[idx_den], add=True)

        def head(h, _):
            att_k = [att_v[h, pl.ds(k * 16, 16)] for k in range(HID // 16)]

            @pl.when(s == 0)
            def _():
                lax.fori_loop(0, NPD, zrow, 0)
                pltpu.sync_copy(stage_den.at[pl.ds(0, NPD)], acc_den)

            plsc.subcore_barrier()
            issue(0, slots[0], h)
            issue(1, slots[1], h)

            def pair(i, _, h=h, att_k=att_k):
                process(i * 2, slots[0], h, att_k)
                process(i * 2 + 1, slots[1], h, att_k)
                return 0

            lax.fori_loop(0, NGRP // 2, pair, 0)
            plsc.subcore_barrier()

            @pl.when(s == 0)
            def _():
                pltpu.sync_copy(acc_den, denp.at[c, h])

            plsc.subcore_barrier()
            return 0

        lax.fori_loop(0, H, head, 0)

    return pl.kernel(
        body,
        out_type=(jax.ShapeDtypeStruct((H * E2P,), _f32),
                  jax.ShapeDtypeStruct((2, H, NPD, HID), _f32)),
        mesh=mesh,
        scratch_types=[
            pltpu.VMEM((NGS, GW), _i32),
            pltpu.VMEM((NGS, GW), _i32),
            pltpu.VMEM((GW,), _i32),
            pltpu.VMEM((GW,), _i32),
            pltpu.VMEM((GW,), _i32),
            pltpu.VMEM((GW,), _i32),
            pltpu.VMEM((GW, HID), _f32),
            pltpu.VMEM((GW, HID), _f32),
            pltpu.VMEM((GW, HID), _f32),
            pltpu.VMEM((GW, HID), _f32),
            pltpu.VMEM((GW,), _f32),
            pltpu.VMEM((GW, HID), _f32),
            pltpu.VMEM((GW,), _i32),
            pltpu.VMEM((H, HID), _f32),
            pltpu.VMEM_SHARED((NPD, HID), _f32),
            pltpu.SemaphoreType.DMA,
            pltpu.SemaphoreType.DMA,
            pltpu.SemaphoreType.DMA,
            pltpu.SemaphoreType.DMA,
        ],
    )


@functools.lru_cache(maxsize=None)
def _sc_aggregate(H):
    mesh = plsc.VectorSubcoreMesh(core_axis_name="c", subcore_axis_name="s")

    def body(xlh, src2d, dst2d, ex_h, outp,
             src_v, dst_v, idxs0, idxs1, ex0, ex1, xl0, xl1,
             acc, sl0, sl1, se0, se1):
        c = lax.axis_index("c")
        s = lax.axis_index("s")
        wid = s * 2 + c
        zv = jnp.zeros((16,), _f32)
        pltpu.sync_copy(src2d.at[pl.ds(wid * NGS, NGS)], src_v)
        pltpu.sync_copy(dst2d.at[pl.ds(wid * NGS, NGS)], dst_v)
        slots = ((idxs0, ex0, xl0, sl0, se0),
                 (idxs1, ex1, xl1, sl1, se1))

        def zrow(i, _):
            for v in range(HID // 16):
                xl0[i, pl.ds(v * 16, 16)] = zv
            return 0

        def issue(g, slot, h):
            idxs, ex_b, xl_b, s_l, s_e = slot
            for j in range(8):
                sv = src_v[g, pl.ds(j * 16, 16)]
                idxs[pl.ds(j * 16, 16)] = sv * H + h
            pltpu.async_copy(xlh.at[idxs], xl_b, s_l)
            pltpu.async_copy(
                ex_h.at[pl.ds(h * E2P + wid * CHS + g * GW, GW)], ex_b, s_e)

        def process(g, slot, h):
            idxs, ex_b, xl_b, s_l, s_e = slot
            pltpu.make_async_copy(xlh.at[idxs], xl_b, s_l).wait()
            pltpu.make_async_copy(
                ex_h.at[pl.ds(h * E2P + wid * CHS + g * GW, GW)],
                ex_b, s_e).wait()
            for sg in range(8):
                exv = ex_b[pl.ds(sg * 16, 16)]

                def row(i, _, exv=exv, sg=sg):
                    e = sg * 16 + i
                    lane = jnp.full((16,), i, _i32)
                    sc = exv.at[lane].get(mode="promise_in_bounds")
                    for v in range(HID // 16):
                        xl_b[e, pl.ds(v * 16, 16)] = (
                            xl_b[e, pl.ds(v * 16, 16)] * sc)
                    return 0

                lax.fori_loop(0, 16, row, 0)
            pltpu.sync_copy(xl_b, acc.at[dst_v.at[g]], add=True)

            @pl.when(g < NGRP - 2)
            def _():
                issue(g + 2, slot, h)

        def head(h, _):
            lax.fori_loop(0, GW, zrow, 0)
            for zoff, zlen in ((0, 128), (128, 128), (256, 128),
                               (384, 128), (512, 120)):
                pltpu.sync_copy(
                    xl0.at[pl.ds(0, zlen)],
                    acc.at[pl.ds(s * ROWS_T + zoff, zlen)])
            plsc.subcore_barrier()
            issue(0, slots[0], h)
            issue(1, slots[1], h)

            def pair(i, _, h=h):
                process(i * 2, slots[0], h)
                process(i * 2 + 1, slots[1], h)
                return 0

            lax.fori_loop(0, NGRP // 2, pair, 0)
            plsc.subcore_barrier()
            pltpu.sync_copy(acc.at[pl.ds(s * ROWS_T, ROWS_T)],
                            outp.at[c, h, pl.ds(s * ROWS_T, ROWS_T)])
            plsc.subcore_barrier()
            return 0

        lax.fori_loop(0, H, head, 0)

    return pl.kernel(
        body,
        out_type=(jax.ShapeDtypeStruct((2, H, NP, HID), _f32),),
        mesh=mesh,
        scratch_types=[
            pltpu.VMEM((NGS, GW), _i32),
            pltpu.VMEM((NGS, GW), _i32),
            pltpu.VMEM((GW,), _i32),
            pltpu.VMEM((GW,), _i32),
            pltpu.VMEM((GW,), _f32),
            pltpu.VMEM((GW,), _f32),
            pltpu.VMEM((GW, HID), _f32),
            pltpu.VMEM((GW, HID), _f32),
            pltpu.VMEM_SHARED((NP, HID), _f32),
            pltpu.SemaphoreType.DMA,
            pltpu.SemaphoreType.DMA,
            pltpu.SemaphoreType.DMA,
            pltpu.SemaphoreType.DMA,
        ],
    )


def _mm2(x, w1, w2):
    npad, k = x.shape
    f = w1.shape[1]
    br = 128
    grid = (npad // br,)

    def body(x_ref, w1_ref, w2_ref, o1_ref, o2_ref):
        xb = x_ref[...]
        o1_ref[...] = jnp.dot(xb, w1_ref[...], preferred_element_type=_f32)
        o2_ref[...] = jnp.dot(xb, w2_ref[...], preferred_element_type=_f32)

    return pl.pallas_call(
        body,
        grid=grid,
        in_specs=[
            pl.BlockSpec((br, k), lambda i: (i, 0)),
            pl.BlockSpec((k, f), lambda i: (0, 0)),
            pl.BlockSpec((k, f), lambda i: (0, 0)),
        ],
        out_specs=[pl.BlockSpec((br, f), lambda i: (i, 0))] * 2,
        out_shape=[jax.ShapeDtypeStruct((npad, f), _f32)] * 2,
    )(x, w1, w2)


def _combine_stats(outp, denp, H):
    f = H * HID
    br = 128
    grid = (NP // br,)

    def body(op_ref, dp_ref, y_ref, st_ref, acc_ref):
        g = pl.program_id(0)
        num = op_ref[0] + op_ref[1]
        den = dp_ref[0] + dp_ref[1]
        yb = num.transpose(1, 0, 2) / (den.T[:, :, None] + 1e-16)
        yb = yb.reshape(br, f)
        y_ref[...] = yb

        @pl.when(g == 0)
        def _():
            acc_ref[...] = jnp.zeros_like(acc_ref)

        acc_ref[0, :] += jnp.sum(yb, axis=0)
        acc_ref[1, :] += jnp.sum(yb * yb, axis=0)

        @pl.when(g == NP // br - 1)
        def _():
            st_ref[...] = acc_ref[...]

    return pl.pallas_call(
        body,
        grid=grid,
        in_specs=[
            pl.BlockSpec((2, H, br, HID), lambda i: (0, 0, i, 0)),
            pl.BlockSpec((2, H, br), lambda i: (0, 0, i)),
        ],
        out_specs=[
            pl.BlockSpec((br, f), lambda i: (i, 0)),
            pl.BlockSpec((8, f), lambda i: (0, 0)),
        ],
        out_shape=[
            jax.ShapeDtypeStruct((NP, f), _f32),
            jax.ShapeDtypeStruct((8, f), _f32),
        ],
        scratch_shapes=[pltpu.VMEM((8, f), _f32)],
    )(outp, denp)


def _bn_block(y, st, g, b):
    mu = st[0, :] * (1.0 / N)
    var = st[1, :] * (1.0 / N) - mu * mu
    inv = lax.rsqrt(var + 1e-5)
    return (y - mu[None, :]) * (inv * g)[None, :] + b[None, :]


def _norm_proj_mm(y, st, bn_g, bn_b, pw, pb, wl, wr):
    f = y.shape[1]
    br = 128
    grid = (NP // br,)

    def body(y_ref, st_ref, g_ref, b_ref, pw_ref, pb_ref, wl_ref, wr_ref,
             xp_ref, xl_ref, xr_ref):
        x1 = jnp.maximum(
            _bn_block(y_ref[...], st_ref[...], g_ref[0], b_ref[0]), 0.0)
        xp = jnp.dot(x1, pw_ref[...], preferred_element_type=_f32) + pb_ref[0]
        xp_ref[...] = xp
        xl_ref[...] = jnp.dot(xp, wl_ref[...], preferred_element_type=_f32)
        xr_ref[...] = jnp.dot(xp, wr_ref[...], preferred_element_type=_f32)

    return pl.pallas_call(
        body,
        grid=grid,
        in_specs=[
            pl.BlockSpec((br, f), lambda i: (i, 0)),
            pl.BlockSpec((8, f), lambda i: (0, 0)),
            pl.BlockSpec((1, f), lambda i: (0, 0)),
            pl.BlockSpec((1, f), lambda i: (0, 0)),
            pl.BlockSpec((f, HID), lambda i: (0, 0)),
            pl.BlockSpec((1, HID), lambda i: (0, 0)),
            pl.BlockSpec((HID, HID), lambda i: (0, 0)),
            pl.BlockSpec((HID, HID), lambda i: (0, 0)),
        ],
        out_specs=[pl.BlockSpec((br, HID), lambda i: (i, 0))] * 3,
        out_shape=[jax.ShapeDtypeStruct((NP, HID), _f32)] * 3,
    )(y, st, bn_g, bn_b, pw, pb, wl, wr)


def _norm_res_mm(y, st, bn_g, bn_b, res, wl, wr):
    br = 128
    grid = (NP // br,)

    def body(y_ref, st_ref, g_ref, b_ref, r_ref, wl_ref, wr_ref,
             xt_ref, xl_ref, xr_ref):
        xt = jnp.maximum(
            _bn_block(y_ref[...], st_ref[...], g_ref[0], b_ref[0]), 0.0)
        xt = xt + r_ref[...]
        xt_ref[...] = xt
        xl_ref[...] = jnp.dot(xt, wl_ref[...], preferred_element_type=_f32)
        xr_ref[...] = jnp.dot(xt, wr_ref[...], preferred_element_type=_f32)

    return pl.pallas_call(
        body,
        grid=grid,
        in_specs=[
            pl.BlockSpec((br, HID), lambda i: (i, 0)),
            pl.BlockSpec((8, HID), lambda i: (0, 0)),
            pl.BlockSpec((1, HID), lambda i: (0, 0)),
            pl.BlockSpec((1, HID), lambda i: (0, 0)),
            pl.BlockSpec((br, HID), lambda i: (i, 0)),
            pl.BlockSpec((HID, HID), lambda i: (0, 0)),
            pl.BlockSpec((HID, HID), lambda i: (0, 0)),
        ],
        out_specs=[pl.BlockSpec((br, HID), lambda i: (i, 0))] * 3,
        out_shape=[jax.ShapeDtypeStruct((NP, HID), _f32)] * 3,
    )(y, st, bn_g, bn_b, res, wl, wr)


def _norm_res_pool(y, st, bn_g, bn_b, res, batch3d, lw, lb):
    br = 128
    grid = (NP // br,)

    def body(y_ref, st_ref, g_ref, b_ref, r_ref, bt_ref, lw_ref, lb_ref,
             o_ref, acc_ref):
        g = pl.program_id(0)
        x5 = jnp.maximum(
            _bn_block(y_ref[...], st_ref[...], g_ref[0], b_ref[0]), 0.0)
        x5 = x5 + r_ref[...]
        bt = bt_ref[0]
        oh = jnp.where(
            bt.reshape(br, 1) == lax.broadcasted_iota(_i32, (br, NG), 1),
            1.0, 0.0)

        @pl.when(g == 0)
        def _():
            acc_ref[...] = jnp.zeros_like(acc_ref)

        acc_ref[...] += lax.dot_general(oh, x5, (((0,), (0,)), ((), ())),
                                        preferred_element_type=_f32)

        @pl.when(g == NP // br - 1)
        def _():
            o_ref[...] = jnp.dot(acc_ref[...], lw_ref[...],
                                 preferred_element_type=_f32) + lb_ref[0]

    return pl.pallas_call(
        body,
        grid=grid,
        in_specs=[
            pl.BlockSpec((br, HID), lambda i: (i, 0)),
            pl.BlockSpec((8, HID), lambda i: (0, 0)),
            pl.BlockSpec((1, HID), lambda i: (0, 0)),
            pl.BlockSpec((1, HID), lambda i: (0, 0)),
            pl.BlockSpec((br, HID), lambda i: (i, 0)),
            pl.BlockSpec((1, 1, br), lambda i: (i, 0, 0)),
            pl.BlockSpec((HID, 128), lambda i: (0, 0)),
            pl.BlockSpec((1, 128), lambda i: (0, 0)),
        ],
        out_specs=[pl.BlockSpec((NG, 128), lambda i: (0, 0))],
        out_shape=[jax.ShapeDtypeStruct((NG, 128), _f32)],
        scratch_shapes=[pltpu.VMEM((NG, HID), _f32)],
    )(y, st, bn_g, bn_b, res, batch3d, lw, lb)


def _gat_layer(xl, xr, src2d, dst2d, att, H):
    xlh = xl.reshape(NP * H, HID)
    xrh = xr.reshape(NP * H, HID)
    ex, denp = _sc_scores(H)(xlh, xrh, src2d, dst2d, att)
    (outp,) = _sc_aggregate(H)(xlh, src2d, dst2d, ex)
    denp = denp.reshape(2, H, NPD * HID)[:, :, :NP]
    return _combine_stats(outp, denp, H)


def kernel(x, edge_index, batch, gat1_Wl, gat1_Wr, gat1_att, gat1_b,
           proj1_W, proj1_b, gat2_Wl, gat2_Wr, gat2_att, gat2_b,
           gat3_Wl, gat3_Wr, gat3_att, gat3_b, gat4_Wl, gat4_Wr, gat4_att,
           gat4_b, gat5_Wl, gat5_Wr, gat5_att, gat5_b, bn1_g, bn1_b,
           bn2_g, bn2_b, bn3_g, bn3_b, bn4_g, bn4_b, bn5_g, bn5_b,
           lin_W, lin_b):
    loop = jnp.arange(N, dtype=jnp.int32)
    src = jnp.concatenate([edge_index[0], loop, jnp.zeros((1,), jnp.int32)])
    dst = jnp.concatenate([edge_index[1], loop, jnp.zeros((1,), jnp.int32)])
    src_p = src[_EDGE_MAP]
    dst_p = dst[_EDGE_MAP]
    src2d = src_p.reshape(E2P // GW, GW)
    dst2d = dst_p.reshape(E2P // GW, GW)
    xp0 = jnp.pad(x, ((0, NP - N), (0, 0)))
    batch3d = jnp.pad(batch, (0, NP - N), constant_values=NG).reshape(
        NP // GW, 1, GW)
    row = lambda v: v.reshape(1, -1)
    lwp = jnp.pad(lin_W, ((0, 0), (0, 128 - NCLS)))
    lbp = jnp.pad(lin_b, (0, 128 - NCLS)).reshape(1, 128)

    xl1, xr1 = _mm2(xp0, gat1_Wl, gat1_Wr)
    y1, st1 = _gat_layer(xl1, xr1, src2d, dst2d, gat1_att, HEADS)
    x1p, xl2, xr2 = _norm_proj_mm(y1, st1, row(bn1_g), row(bn1_b),
                                  proj1_W, row(proj1_b), gat2_Wl, gat2_Wr)

    y2, st2 = _gat_layer(xl2, xr2, src2d, dst2d, gat2_att, 1)
    x2, xl3, xr3 = _norm_res_mm(y2, st2, row(bn2_g), row(bn2_b), x1p,
                                gat3_Wl, gat3_Wr)
    y3, st3 = _gat_layer(xl3, xr3, src2d, dst2d, gat3_att, 1)
    x3, xl4, xr4 = _norm_res_mm(y3, st3, row(bn3_g), row(bn3_b), x2,
                                gat4_Wl, gat4_Wr)
    y4, st4 = _gat_layer(xl4, xr4, src2d, dst2d, gat4_att, 1)
    x4, xl5, xr5 = _norm_res_mm(y4, st4, row(bn4_g), row(bn4_b), x3,
                                gat5_Wl, gat5_Wr)

    y5, st5 = _gat_layer(xl5, xr5, src2d, dst2d, gat5_att, 1)
    (outp,) = _norm_res_pool(y5, st5, row(bn5_g), row(bn5_b), x4,
                             batch3d, lwp, lbp)
    return outp[:, :NCLS]


# speedup vs baseline: 9.7491x; 1.0725x over previous
import functools

import jax
import jax.numpy as jnp
from jax import lax
from jax.experimental import pallas as pl
from jax.experimental.pallas import tpu as pltpu
from jax.experimental.pallas import tpu_sc as plsc

N = 10000
E = 160000
E2 = E + N
DIN = 128
HID = 128
HEADS = 4
NCLS = 7
NG = 64

NP = 10112
GW = 128
NW = 32
NGRP = 42
NGS = 48
CHS = NGS * GW
E2P = NW * CHS
CNT_LO = E2 // NW
CNT_HI = CNT_LO + 1
ROWS_T = NP // 16
NPD = 80

_f32 = jnp.float32
_i32 = jnp.int32


def _edge_map():
    import numpy as np
    m = np.full((E2P,), E2, np.int32)
    pos = 0
    for w in range(NW):
        c = CNT_HI if w < 16 else CNT_LO
        m[w * CHS:w * CHS + c] = np.arange(pos, pos + c, dtype=np.int32)
        pos += c
    assert pos == E2
    return m


_EDGE_MAP = _edge_map()


@functools.lru_cache(maxsize=None)
def _sc_scores(H):
    mesh = plsc.VectorSubcoreMesh(core_axis_name="c", subcore_axis_name="s")

    def body(xlh, xrh, src2d, dst2d, att_h, ex_h, denp,
             src_v, dst_v, idxs0, idxs1, idxd0, idxd1, xl0, xl1, xr0, xr1,
             ex_buf, sd0, sd1, id0, id1, att_v, acc_den,
             sl0, sl1, sr0, sr1, sdm0, sdm1):
        c = lax.axis_index("c")
        s = lax.axis_index("s")
        wid = s * 2 + c
        cnt = jnp.where(wid < 16, CNT_HI, CNT_LO)
        pltpu.sync_copy(src2d.at[pl.ds(wid * NGS, NGS)], src_v)
        pltpu.sync_copy(dst2d.at[pl.ds(wid * NGS, NGS)], dst_v)
        pltpu.sync_copy(att_h, att_v)
        iota = lax.iota(_i32, 16)
        zv = jnp.zeros((16,), _f32)
        slots = ((idxs0, idxd0, xl0, xr0, sl0, sr0, sd0, id0, sdm0),
                 (idxs1, idxd1, xl1, xr1, sl1, sr1, sd1, id1, sdm1))

        def zrow(i, _):
            for v in range(HID // 16):
                sd0[i, pl.ds(v * 16, 16)] = zv
            return 0

        def issue(g, slot, h):
            idxs, idxd, xl_b, xr_b, s_l, s_r = slot[:6]
            for j in range(8):
                sv = src_v[g, pl.ds(j * 16, 16)]
                idxs[pl.ds(j * 16, 16)] = sv * H + h
                dv = dst_v[g, pl.ds(j * 16, 16)]
                idxd[pl.ds(j * 16, 16)] = dv * H + h
            pltpu.async_copy(xlh.at[idxs], xl_b, s_l)
            pltpu.async_copy(xrh.at[idxd], xr_b, s_r)

        def process(g, slot, h, att_k):
            idxs, idxd, xl_b, xr_b, s_l, s_r, sd_b, id_b, s_d = slot
            pltpu.make_async_copy(xlh.at[idxs], xl_b, s_l).wait()
            pltpu.make_async_copy(xrh.at[idxd], xr_b, s_r).wait()
            for sg in range(8):

                def edge(i, sv, sg=sg):
                    e = sg * 16 + i
                    acc = jnp.zeros((16,), _f32)
                    for k in range(HID // 16):
                        m = (xl_b[e, pl.ds(k * 16, 16)]
                             + xr_b[e, pl.ds(k * 16, 16)])
                        m = jnp.maximum(m, m * 0.2)
                        acc = acc + m * att_k[k]
                    for b in (1, 2, 4, 8):
                        perm = jnp.bitwise_xor(iota, b)
                        acc = acc + acc.at[perm].get(
                            mode="promise_in_bounds")
                    return jnp.where(iota == i, acc, sv)

                score = lax.fori_loop(0, 16, edge, jnp.zeros((16,), _f32))
                le = iota + (g * GW + sg * 16)
                exv = jnp.where(le < cnt, jnp.exp(score), 0.0)
                ex_buf[pl.ds(sg * 16, 16)] = exv
            pltpu.sync_copy(
                ex_buf, ex_h.at[pl.ds(h * E2P + wid * CHS + g * GW, GW)])
            @pl.when(g < NGRP - 2)
            def _():
                issue(g + 2, slot, h)
            @pl.when(g >= 2)
            def _():
                pltpu.make_async_copy(sd_b, acc_den.at[id_b], s_d).wait()

            for j in range(8):
                dv = dst_v[g, pl.ds(j * 16, 16)]
                id_b[pl.ds(j * 16, 16)] = jnp.right_shift(dv, 7)
            for sg in range(8):
                exv = ex_buf[pl.ds(sg * 16, 16)]
                dmod = jnp.bitwise_and(dst_v[g, pl.ds(sg * 16, 16)], 127)

                def row(i, _, exv=exv, dmod=dmod, sg=sg):
                    e = sg * 16 + i
                    lane = jnp.full((16,), i, _i32)
                    sc = exv.at[lane].get(mode="promise_in_bounds")
                    dm = dmod.at[lane].get(mode="promise_in_bounds")
                    for v in range(HID // 16):
                        sd_b[e, pl.ds(v * 16, 16)] = jnp.where(
                            iota + (v * 16) == dm, sc, 0.0)
                    return 0

                lax.fori_loop(0, 16, row, 0)
            pltpu.async_copy(sd_b, acc_den.at[id_b], s_d, add=True)

        def head(h, _):
            att_k = [att_v[h, pl.ds(k * 16, 16)] for k in range(HID // 16)]

            @pl.when(s == 0)
            def _():
                lax.fori_loop(0, NPD, zrow, 0)
                pltpu.sync_copy(sd0.at[pl.ds(0, NPD)], acc_den)

            plsc.subcore_barrier()
            issue(0, slots[0], h)
            issue(1, slots[1], h)

            def pair(i, _, h=h, att_k=att_k):
                process(i * 2, slots[0], h, att_k)
                process(i * 2 + 1, slots[1], h, att_k)
                return 0

            lax.fori_loop(0, NGRP // 2, pair, 0)
            pltpu.make_async_copy(sd0, acc_den.at[id0], sdm0).wait()
            pltpu.make_async_copy(sd1, acc_den.at[id1], sdm1).wait()
            plsc.subcore_barrier()

            @pl.when(s == 0)
            def _():
                pltpu.sync_copy(acc_den, denp.at[c, h])

            plsc.subcore_barrier()
            return 0

        lax.fori_loop(0, H, head, 0)

    return pl.kernel(
        body,
        out_type=(jax.ShapeDtypeStruct((H * E2P,), _f32),
                  jax.ShapeDtypeStruct((2, H, NPD, HID), _f32)),
        mesh=mesh,
        scratch_types=[
            pltpu.VMEM((NGS, GW), _i32),
            pltpu.VMEM((NGS, GW), _i32),
            pltpu.VMEM((GW,), _i32),
            pltpu.VMEM((GW,), _i32),
            pltpu.VMEM((GW,), _i32),
            pltpu.VMEM((GW,), _i32),
            pltpu.VMEM((GW, HID), _f32),
            pltpu.VMEM((GW, HID), _f32),
            pltpu.VMEM((GW, HID), _f32),
            pltpu.VMEM((GW, HID), _f32),
            pltpu.VMEM((GW,), _f32),
            pltpu.VMEM((GW, HID), _f32),
            pltpu.VMEM((GW, HID), _f32),
            pltpu.VMEM((GW,), _i32),
            pltpu.VMEM((GW,), _i32),
            pltpu.VMEM((H, HID), _f32),
            pltpu.VMEM_SHARED((NPD, HID), _f32),
            pltpu.SemaphoreType.DMA,
            pltpu.SemaphoreType.DMA,
            pltpu.SemaphoreType.DMA,
            pltpu.SemaphoreType.DMA,
            pltpu.SemaphoreType.DMA,
            pltpu.SemaphoreType.DMA,
        ],
    )


@functools.lru_cache(maxsize=None)
def _sc_aggregate(H):
    mesh = plsc.VectorSubcoreMesh(core_axis_name="c", subcore_axis_name="s")

    def body(xlh, src2d, dst2d, ex_h, outp,
             src_v, dst_v, idxs0, idxs1, ex0, ex1, xl0, xl1,
             acc, sl0, sl1, se0, se1):
        c = lax.axis_index("c")
        s = lax.axis_index("s")
        wid = s * 2 + c
        zv = jnp.zeros((16,), _f32)
        pltpu.sync_copy(src2d.at[pl.ds(wid * NGS, NGS)], src_v)
        pltpu.sync_copy(dst2d.at[pl.ds(wid * NGS, NGS)], dst_v)
        slots = ((idxs0, ex0, xl0, sl0, se0),
                 (idxs1, ex1, xl1, sl1, se1))

        def zrow(i, _):
            for v in range(HID // 16):
                xl0[i, pl.ds(v * 16, 16)] = zv
            return 0

        def issue(g, slot, h):
            idxs, ex_b, xl_b, s_l, s_e = slot
            for j in range(8):
                sv = src_v[g, pl.ds(j * 16, 16)]
                idxs[pl.ds(j * 16, 16)] = sv * H + h
            pltpu.async_copy(xlh.at[idxs], xl_b, s_l)
            pltpu.async_copy(
                ex_h.at[pl.ds(h * E2P + wid * CHS + g * GW, GW)], ex_b, s_e)

        def process(g, slot, h):
            idxs, ex_b, xl_b, s_l, s_e = slot
            pltpu.make_async_copy(xlh.at[idxs], xl_b, s_l).wait()
            pltpu.make_async_copy(
                ex_h.at[pl.ds(h * E2P + wid * CHS + g * GW, GW)],
                ex_b, s_e).wait()
            for sg in range(8):
                exv = ex_b[pl.ds(sg * 16, 16)]

                def row(i, _, exv=exv, sg=sg):
                    e = sg * 16 + i
                    lane = jnp.full((16,), i, _i32)
                    sc = exv.at[lane].get(mode="promise_in_bounds")
                    for v in range(HID // 16):
                        xl_b[e, pl.ds(v * 16, 16)] = (
                            xl_b[e, pl.ds(v * 16, 16)] * sc)
                    return 0

                lax.fori_loop(0, 16, row, 0)
            pltpu.sync_copy(xl_b, acc.at[dst_v.at[g]], add=True)

            @pl.when(g < NGRP - 2)
            def _():
                issue(g + 2, slot, h)

        def head(h, _):
            lax.fori_loop(0, GW, zrow, 0)
            for zoff, zlen in ((0, 128), (128, 128), (256, 128),
                               (384, 128), (512, 120)):
                pltpu.sync_copy(
                    xl0.at[pl.ds(0, zlen)],
                    acc.at[pl.ds(s * ROWS_T + zoff, zlen)])
            plsc.subcore_barrier()
            issue(0, slots[0], h)
            issue(1, slots[1], h)

            def pair(i, _, h=h):
                process(i * 2, slots[0], h)
                process(i * 2 + 1, slots[1], h)
                return 0

            lax.fori_loop(0, NGRP // 2, pair, 0)
            plsc.subcore_barrier()
            pltpu.sync_copy(acc.at[pl.ds(s * ROWS_T, ROWS_T)],
                            outp.at[c, h, pl.ds(s * ROWS_T, ROWS_T)])
            plsc.subcore_barrier()
            return 0

        lax.fori_loop(0, H, head, 0)

    return pl.kernel(
        body,
        out_type=(jax.ShapeDtypeStruct((2, H, NP, HID), _f32),),
        mesh=mesh,
        scratch_types=[
            pltpu.VMEM((NGS, GW), _i32),
            pltpu.VMEM((NGS, GW), _i32),
            pltpu.VMEM((GW,), _i32),
            pltpu.VMEM((GW,), _i32),
            pltpu.VMEM((GW,), _f32),
            pltpu.VMEM((GW,), _f32),
            pltpu.VMEM((GW, HID), _f32),
            pltpu.VMEM((GW, HID), _f32),
            pltpu.VMEM_SHARED((NP, HID), _f32),
            pltpu.SemaphoreType.DMA,
            pltpu.SemaphoreType.DMA,
            pltpu.SemaphoreType.DMA,
            pltpu.SemaphoreType.DMA,
        ],
    )


def _mm2(x, w1, w2):
    npad, k = x.shape
    f = w1.shape[1]
    br = 128
    grid = (npad // br,)

    def body(x_ref, w1_ref, w2_ref, o1_ref, o2_ref):
        xb = x_ref[...]
        o1_ref[...] = jnp.dot(xb, w1_ref[...], preferred_element_type=_f32)
        o2_ref[...] = jnp.dot(xb, w2_ref[...], preferred_element_type=_f32)

    return pl.pallas_call(
        body,
        grid=grid,
        in_specs=[
            pl.BlockSpec((br, k), lambda i: (i, 0)),
            pl.BlockSpec((k, f), lambda i: (0, 0)),
            pl.BlockSpec((k, f), lambda i: (0, 0)),
        ],
        out_specs=[pl.BlockSpec((br, f), lambda i: (i, 0))] * 2,
        out_shape=[jax.ShapeDtypeStruct((npad, f), _f32)] * 2,
    )(x, w1, w2)


def _combine_stats(outp, denp, H):
    f = H * HID
    br = 128
    grid = (NP // br,)

    def body(op_ref, dp_ref, y_ref, st_ref, acc_ref):
        g = pl.program_id(0)
        num = op_ref[0] + op_ref[1]
        den = dp_ref[0] + dp_ref[1]
        yb = num.transpose(1, 0, 2) / (den.T[:, :, None] + 1e-16)
        yb = yb.reshape(br, f)
        y_ref[...] = yb

        @pl.when(g == 0)
        def _():
            acc_ref[...] = jnp.zeros_like(acc_ref)

        acc_ref[0, :] += jnp.sum(yb, axis=0)
        acc_ref[1, :] += jnp.sum(yb * yb, axis=0)

        @pl.when(g == NP // br - 1)
        def _():
            st_ref[...] = acc_ref[...]

    return pl.pallas_call(
        body,
        grid=grid,
        in_specs=[
            pl.BlockSpec((2, H, br, HID), lambda i: (0, 0, i, 0)),
            pl.BlockSpec((2, H, br), lambda i: (0, 0, i)),
        ],
        out_specs=[
            pl.BlockSpec((br, f), lambda i: (i, 0)),
            pl.BlockSpec((8, f), lambda i: (0, 0)),
        ],
        out_shape=[
            jax.ShapeDtypeStruct((NP, f), _f32),
            jax.ShapeDtypeStruct((8, f), _f32),
        ],
        scratch_shapes=[pltpu.VMEM((8, f), _f32)],
    )(outp, denp)


def _bn_block(y, st, g, b):
    mu = st[0, :] * (1.0 / N)
    var = st[1, :] * (1.0 / N) - mu * mu
    inv = lax.rsqrt(var + 1e-5)
    return (y - mu[None, :]) * (inv * g)[None, :] + b[None, :]


def _norm_proj_mm(y, st, bn_g, bn_b, pw, pb, wl, wr):
    f = y.shape[1]
    br = 128
    grid = (NP // br,)

    def body(y_ref, st_ref, g_ref, b_ref, pw_ref, pb_ref, wl_ref, wr_ref,
             xp_ref, xl_ref, xr_ref):
        x1 = jnp.maximum(
            _bn_block(y_ref[...], st_ref[...], g_ref[0], b_ref[0]), 0.0)
        xp = jnp.dot(x1, pw_ref[...], preferred_element_type=_f32) + pb_ref[0]
        xp_ref[...] = xp
        xl_ref[...] = jnp.dot(xp, wl_ref[...], preferred_element_type=_f32)
        xr_ref[...] = jnp.dot(xp, wr_ref[...], preferred_element_type=_f32)

    return pl.pallas_call(
        body,
        grid=grid,
        in_specs=[
            pl.BlockSpec((br, f), lambda i: (i, 0)),
            pl.BlockSpec((8, f), lambda i: (0, 0)),
            pl.BlockSpec((1, f), lambda i: (0, 0)),
            pl.BlockSpec((1, f), lambda i: (0, 0)),
            pl.BlockSpec((f, HID), lambda i: (0, 0)),
            pl.BlockSpec((1, HID), lambda i: (0, 0)),
            pl.BlockSpec((HID, HID), lambda i: (0, 0)),
            pl.BlockSpec((HID, HID), lambda i: (0, 0)),
        ],
        out_specs=[pl.BlockSpec((br, HID), lambda i: (i, 0))] * 3,
        out_shape=[jax.ShapeDtypeStruct((NP, HID), _f32)] * 3,
    )(y, st, bn_g, bn_b, pw, pb, wl, wr)


def _norm_res_mm(y, st, bn_g, bn_b, res, wl, wr):
    br = 128
    grid = (NP // br,)

    def body(y_ref, st_ref, g_ref, b_ref, r_ref, wl_ref, wr_ref,
             xt_ref, xl_ref, xr_ref):
        xt = jnp.maximum(
            _bn_block(y_ref[...], st_ref[...], g_ref[0], b_ref[0]), 0.0)
        xt = xt + r_ref[...]
        xt_ref[...] = xt
        xl_ref[...] = jnp.dot(xt, wl_ref[...], preferred_element_type=_f32)
        xr_ref[...] = jnp.dot(xt, wr_ref[...], preferred_element_type=_f32)

    return pl.pallas_call(
        body,
        grid=grid,
        in_specs=[
            pl.BlockSpec((br, HID), lambda i: (i, 0)),
            pl.BlockSpec((8, HID), lambda i: (0, 0)),
            pl.BlockSpec((1, HID), lambda i: (0, 0)),
            pl.BlockSpec((1, HID), lambda i: (0, 0)),
            pl.BlockSpec((br, HID), lambda i: (i, 0)),
            pl.BlockSpec((HID, HID), lambda i: (0, 0)),
            pl.BlockSpec((HID, HID), lambda i: (0, 0)),
        ],
        out_specs=[pl.BlockSpec((br, HID), lambda i: (i, 0))] * 3,
        out_shape=[jax.ShapeDtypeStruct((NP, HID), _f32)] * 3,
    )(y, st, bn_g, bn_b, res, wl, wr)


def _norm_res_pool(y, st, bn_g, bn_b, res, batch3d, lw, lb):
    br = 128
    grid = (NP // br,)

    def body(y_ref, st_ref, g_ref, b_ref, r_ref, bt_ref, lw_ref, lb_ref,
             o_ref, acc_ref):
        g = pl.program_id(0)
        x5 = jnp.maximum(
            _bn_block(y_ref[...], st_ref[...], g_ref[0], b_ref[0]), 0.0)
        x5 = x5 + r_ref[...]
        bt = bt_ref[0]
        oh = jnp.where(
            bt.reshape(br, 1) == lax.broadcasted_iota(_i32, (br, NG), 1),
            1.0, 0.0)

        @pl.when(g == 0)
        def _():
            acc_ref[...] = jnp.zeros_like(acc_ref)

        acc_ref[...] += lax.dot_general(oh, x5, (((0,), (0,)), ((), ())),
                                        preferred_element_type=_f32)

        @pl.when(g == NP // br - 1)
        def _():
            o_ref[...] = jnp.dot(acc_ref[...], lw_ref[...],
                                 preferred_element_type=_f32) + lb_ref[0]

    return pl.pallas_call(
        body,
        grid=grid,
        in_specs=[
            pl.BlockSpec((br, HID), lambda i: (i, 0)),
            pl.BlockSpec((8, HID), lambda i: (0, 0)),
            pl.BlockSpec((1, HID), lambda i: (0, 0)),
            pl.BlockSpec((1, HID), lambda i: (0, 0)),
            pl.BlockSpec((br, HID), lambda i: (i, 0)),
            pl.BlockSpec((1, 1, br), lambda i: (i, 0, 0)),
            pl.BlockSpec((HID, 128), lambda i: (0, 0)),
            pl.BlockSpec((1, 128), lambda i: (0, 0)),
        ],
        out_specs=[pl.BlockSpec((NG, 128), lambda i: (0, 0))],
        out_shape=[jax.ShapeDtypeStruct((NG, 128), _f32)],
        scratch_shapes=[pltpu.VMEM((NG, HID), _f32)],
    )(y, st, bn_g, bn_b, res, batch3d, lw, lb)


def _gat_layer(xl, xr, src2d, dst2d, att, H):
    xlh = xl.reshape(NP * H, HID)
    xrh = xr.reshape(NP * H, HID)
    ex, denp = _sc_scores(H)(xlh, xrh, src2d, dst2d, att)
    (outp,) = _sc_aggregate(H)(xlh, src2d, dst2d, ex)
    denp = denp.reshape(2, H, NPD * HID)[:, :, :NP]
    return _combine_stats(outp, denp, H)


def kernel(x, edge_index, batch, gat1_Wl, gat1_Wr, gat1_att, gat1_b,
           proj1_W, proj1_b, gat2_Wl, gat2_Wr, gat2_att, gat2_b,
           gat3_Wl, gat3_Wr, gat3_att, gat3_b, gat4_Wl, gat4_Wr, gat4_att,
           gat4_b, gat5_Wl, gat5_Wr, gat5_att, gat5_b, bn1_g, bn1_b,
           bn2_g, bn2_b, bn3_g, bn3_b, bn4_g, bn4_b, bn5_g, bn5_b,
           lin_W, lin_b):
    loop = jnp.arange(N, dtype=jnp.int32)
    src = jnp.concatenate([edge_index[0], loop, jnp.zeros((1,), jnp.int32)])
    dst = jnp.concatenate([edge_index[1], loop, jnp.zeros((1,), jnp.int32)])
    src_p = src[_EDGE_MAP]
    dst_p = dst[_EDGE_MAP]
    src2d = src_p.reshape(E2P // GW, GW)
    dst2d = dst_p.reshape(E2P // GW, GW)
    xp0 = jnp.pad(x, ((0, NP - N), (0, 0)))
    batch3d = jnp.pad(batch, (0, NP - N), constant_values=NG).reshape(
        NP // GW, 1, GW)
    row = lambda v: v.reshape(1, -1)
    lwp = jnp.pad(lin_W, ((0, 0), (0, 128 - NCLS)))
    lbp = jnp.pad(lin_b, (0, 128 - NCLS)).reshape(1, 128)

    xl1, xr1 = _mm2(xp0, gat1_Wl, gat1_Wr)
    y1, st1 = _gat_layer(xl1, xr1, src2d, dst2d, gat1_att, HEADS)
    x1p, xl2, xr2 = _norm_proj_mm(y1, st1, row(bn1_g), row(bn1_b),
                                  proj1_W, row(proj1_b), gat2_Wl, gat2_Wr)

    y2, st2 = _gat_layer(xl2, xr2, src2d, dst2d, gat2_att, 1)
    x2, xl3, xr3 = _norm_res_mm(y2, st2, row(bn2_g), row(bn2_b), x1p,
                                gat3_Wl, gat3_Wr)
    y3, st3 = _gat_layer(xl3, xr3, src2d, dst2d, gat3_att, 1)
    x3, xl4, xr4 = _norm_res_mm(y3, st3, row(bn3_g), row(bn3_b), x2,
                                gat4_Wl, gat4_Wr)
    y4, st4 = _gat_layer(xl4, xr4, src2d, dst2d, gat4_att, 1)
    x4, xl5, xr5 = _norm_res_mm(y4, st4, row(bn4_g), row(bn4_b), x3,
                                gat5_Wl, gat5_Wr)

    y5, st5 = _gat_layer(xl5, xr5, src2d, dst2d, gat5_att, 1)
    (outp,) = _norm_res_pool(y5, st5, row(bn5_g), row(bn5_b), x4,
                             batch3d, lwp, lbp)
    return outp[:, :NCLS]


# speedup vs baseline: 9.8007x; 1.0053x over previous
import functools

import jax
import jax.numpy as jnp
from jax import lax
from jax.experimental import pallas as pl
from jax.experimental.pallas import tpu as pltpu
from jax.experimental.pallas import tpu_sc as plsc

N = 10000
E = 160000
E2 = E + N
DIN = 128
HID = 128
HEADS = 4
NCLS = 7
NG = 64

NP = 10112
GW = 128
NW = 32
NGRP = 42
NGS = 48
CHS = NGS * GW
E2P = NW * CHS
CNT_LO = E2 // NW
CNT_HI = CNT_LO + 1
ROWS_T = NP // 16
NPD = 80

_f32 = jnp.float32
_i32 = jnp.int32


def _edge_map():
    import numpy as np
    m = np.full((E2P,), E2, np.int32)
    pos = 0
    for w in range(NW):
        c = CNT_HI if w < 16 else CNT_LO
        m[w * CHS:w * CHS + c] = np.arange(pos, pos + c, dtype=np.int32)
        pos += c
    assert pos == E2
    return m


_EDGE_MAP = _edge_map()


@functools.lru_cache(maxsize=None)
def _sc_scores(H):
    mesh = plsc.VectorSubcoreMesh(core_axis_name="c", subcore_axis_name="s")

    def body(xlh, xrh, src2d, dst2d, att_h, ex_h, denp,
             src_v, dst_v, idxs0, idxs1, idxd0, idxd1, xl0, xl1, xr0, xr1,
             ex0, ex1, sd0, sd1, id0, id1, att_v, acc_den,
             sl0, sl1, sr0, sr1, sdm0, sdm1, sx0, sx1):
        c = lax.axis_index("c")
        s = lax.axis_index("s")
        wid = s * 2 + c
        cnt = jnp.where(wid < 16, CNT_HI, CNT_LO)
        pltpu.sync_copy(src2d.at[pl.ds(wid * NGS, NGS)], src_v)
        pltpu.sync_copy(dst2d.at[pl.ds(wid * NGS, NGS)], dst_v)
        pltpu.sync_copy(att_h, att_v)
        iota = lax.iota(_i32, 16)
        zv = jnp.zeros((16,), _f32)
        slots = ((idxs0, idxd0, xl0, xr0, sl0, sr0, sd0, id0, sdm0, ex0,
                  sx0),
                 (idxs1, idxd1, xl1, xr1, sl1, sr1, sd1, id1, sdm1, ex1,
                  sx1))

        def zrow(i, _):
            for v in range(HID // 16):
                sd0[i, pl.ds(v * 16, 16)] = zv
            return 0

        def issue(g, slot, h):
            idxs, idxd, xl_b, xr_b, s_l, s_r = slot[:6]
            for j in range(8):
                sv = src_v[g, pl.ds(j * 16, 16)]
                idxs[pl.ds(j * 16, 16)] = sv * H + h
                dv = dst_v[g, pl.ds(j * 16, 16)]
                idxd[pl.ds(j * 16, 16)] = dv * H + h
            pltpu.async_copy(xlh.at[idxs], xl_b, s_l)
            pltpu.async_copy(xrh.at[idxd], xr_b, s_r)

        def process(g, slot, h, att_k):
            (idxs, idxd, xl_b, xr_b, s_l, s_r, sd_b, id_b, s_d, ex_b,
             s_x) = slot
            pltpu.make_async_copy(xlh.at[idxs], xl_b, s_l).wait()
            pltpu.make_async_copy(xrh.at[idxd], xr_b, s_r).wait()
            for sg in range(8):

                def edge(i, sv, sg=sg):
                    e = sg * 16 + i
                    acc = jnp.zeros((16,), _f32)
                    for k in range(HID // 16):
                        m = (xl_b[e, pl.ds(k * 16, 16)]
                             + xr_b[e, pl.ds(k * 16, 16)])
                        m = jnp.maximum(m, m * 0.2)
                        acc = acc + m * att_k[k]
                    for b in (1, 2, 4, 8):
                        perm = jnp.bitwise_xor(iota, b)
                        acc = acc + acc.at[perm].get(
                            mode="promise_in_bounds")
                    return jnp.where(iota == i, acc, sv)

                score = lax.fori_loop(0, 16, edge, jnp.zeros((16,), _f32))
                le = iota + (g * GW + sg * 16)
                exv = jnp.where(le < cnt, jnp.exp(score), 0.0)

                if sg == 0:
                    @pl.when(g >= 2)
                    def _():
                        pltpu.make_async_copy(
                            ex_b,
                            ex_h.at[pl.ds(
                                h * E2P + wid * CHS + (g - 2) * GW, GW)],
                            s_x).wait()

                ex_b[pl.ds(sg * 16, 16)] = exv
            pltpu.async_copy(
                ex_b, ex_h.at[pl.ds(h * E2P + wid * CHS + g * GW, GW)], s_x)
            @pl.when(g < NGRP - 2)
            def _():
                issue(g + 2, slot, h)
            @pl.when(g >= 2)
            def _():
                pltpu.make_async_copy(sd_b, acc_den.at[id_b], s_d).wait()

            for j in range(8):
                dv = dst_v[g, pl.ds(j * 16, 16)]
                id_b[pl.ds(j * 16, 16)] = jnp.right_shift(dv, 7)
            for sg in range(8):
                exv = ex_b[pl.ds(sg * 16, 16)]
                dmod = jnp.bitwise_and(dst_v[g, pl.ds(sg * 16, 16)], 127)

                def row(i, _, exv=exv, dmod=dmod, sg=sg):
                    e = sg * 16 + i
                    lane = jnp.full((16,), i, _i32)
                    sc = exv.at[lane].get(mode="promise_in_bounds")
                    dm = dmod.at[lane].get(mode="promise_in_bounds")
                    for v in range(HID // 16):
                        sd_b[e, pl.ds(v * 16, 16)] = jnp.where(
                            iota + (v * 16) == dm, sc, 0.0)
                    return 0

                lax.fori_loop(0, 16, row, 0)
            pltpu.async_copy(sd_b, acc_den.at[id_b], s_d, add=True)

        def head(h, _):
            att_k = [att_v[h, pl.ds(k * 16, 16)] for k in range(HID // 16)]

            @pl.when(s == 0)
            def _():
                lax.fori_loop(0, NPD, zrow, 0)
                pltpu.sync_copy(sd0.at[pl.ds(0, NPD)], acc_den)

            plsc.subcore_barrier()
            issue(0, slots[0], h)
            issue(1, slots[1], h)

            def pair(i, _, h=h, att_k=att_k):
                process(i * 2, slots[0], h, att_k)
                process(i * 2 + 1, slots[1], h, att_k)
                return 0

            lax.fori_loop(0, NGRP // 2, pair, 0)
            pltpu.make_async_copy(sd0, acc_den.at[id0], sdm0).wait()
            pltpu.make_async_copy(sd1, acc_den.at[id1], sdm1).wait()
            pltpu.make_async_copy(
                ex0, ex_h.at[pl.ds(h * E2P + wid * CHS + (NGRP - 2) * GW,
                                   GW)], sx0).wait()
            pltpu.make_async_copy(
                ex1, ex_h.at[pl.ds(h * E2P + wid * CHS + (NGRP - 1) * GW,
                                   GW)], sx1).wait()
            plsc.subcore_barrier()

            @pl.when(s == 0)
            def _():
                pltpu.sync_copy(acc_den, denp.at[c, h])

            plsc.subcore_barrier()
            return 0

        lax.fori_loop(0, H, head, 0)

    return pl.kernel(
        body,
        out_type=(jax.ShapeDtypeStruct((H * E2P,), _f32),
                  jax.ShapeDtypeStruct((2, H, NPD, HID), _f32)),
        mesh=mesh,
        scratch_types=[
            pltpu.VMEM((NGS, GW), _i32),
            pltpu.VMEM((NGS, GW), _i32),
            pltpu.VMEM((GW,), _i32),
            pltpu.VMEM((GW,), _i32),
            pltpu.VMEM((GW,), _i32),
            pltpu.VMEM((GW,), _i32),
            pltpu.VMEM((GW, HID), _f32),
            pltpu.VMEM((GW, HID), _f32),
            pltpu.VMEM((GW, HID), _f32),
            pltpu.VMEM((GW, HID), _f32),
            pltpu.VMEM((GW,), _f32),
            pltpu.VMEM((GW,), _f32),
            pltpu.VMEM((GW, HID), _f32),
            pltpu.VMEM((GW, HID), _f32),
            pltpu.VMEM((GW,), _i32),
            pltpu.VMEM((GW,), _i32),
            pltpu.VMEM((H, HID), _f32),
            pltpu.VMEM_SHARED((NPD, HID), _f32),
            pltpu.SemaphoreType.DMA,
            pltpu.SemaphoreType.DMA,
            pltpu.SemaphoreType.DMA,
            pltpu.SemaphoreType.DMA,
            pltpu.SemaphoreType.DMA,
            pltpu.SemaphoreType.DMA,
            pltpu.SemaphoreType.DMA,
            pltpu.SemaphoreType.DMA,
        ],
    )


@functools.lru_cache(maxsize=None)
def _sc_aggregate(H):
    mesh = plsc.VectorSubcoreMesh(core_axis_name="c", subcore_axis_name="s")

    def body(xlh, src2d, dst2d, ex_h, outp,
             src_v, dst_v, idxs0, idxs1, ex0, ex1, xl0, xl1,
             acc, sl0, sl1, se0, se1):
        c = lax.axis_index("c")
        s = lax.axis_index("s")
        wid = s * 2 + c
        zv = jnp.zeros((16,), _f32)
        pltpu.sync_copy(src2d.at[pl.ds(wid * NGS, NGS)], src_v)
        pltpu.sync_copy(dst2d.at[pl.ds(wid * NGS, NGS)], dst_v)
        slots = ((idxs0, ex0, xl0, sl0, se0),
                 (idxs1, ex1, xl1, sl1, se1))

        def zrow(i, _):
            for v in range(HID // 16):
                xl0[i, pl.ds(v * 16, 16)] = zv
            return 0

        def issue(g, slot, h):
            idxs, ex_b, xl_b, s_l, s_e = slot
            for j in range(8):
                sv = src_v[g, pl.ds(j * 16, 16)]
                idxs[pl.ds(j * 16, 16)] = sv * H + h
            pltpu.async_copy(xlh.at[idxs], xl_b, s_l)
            pltpu.async_copy(
                ex_h.at[pl.ds(h * E2P + wid * CHS + g * GW, GW)], ex_b, s_e)

        def process(g, slot, h):
            idxs, ex_b, xl_b, s_l, s_e = slot
            pltpu.make_async_copy(xlh.at[idxs], xl_b, s_l).wait()
            pltpu.make_async_copy(
                ex_h.at[pl.ds(h * E2P + wid * CHS + g * GW, GW)],
                ex_b, s_e).wait()
            for sg in range(8):
                exv = ex_b[pl.ds(sg * 16, 16)]

                def row(i, _, exv=exv, sg=sg):
                    e = sg * 16 + i
                    lane = jnp.full((16,), i, _i32)
                    sc = exv.at[lane].get(mode="promise_in_bounds")
                    for v in range(HID // 16):
                        xl_b[e, pl.ds(v * 16, 16)] = (
                            xl_b[e, pl.ds(v * 16, 16)] * sc)
                    return 0

                lax.fori_loop(0, 16, row, 0)
            pltpu.sync_copy(xl_b, acc.at[dst_v.at[g]], add=True)

            @pl.when(g < NGRP - 2)
            def _():
                issue(g + 2, slot, h)

        def head(h, _):
            lax.fori_loop(0, GW, zrow, 0)
            for zoff, zlen in ((0, 128), (128, 128), (256, 128),
                               (384, 128), (512, 120)):
                pltpu.sync_copy(
                    xl0.at[pl.ds(0, zlen)],
                    acc.at[pl.ds(s * ROWS_T + zoff, zlen)])
            plsc.subcore_barrier()
            issue(0, slots[0], h)
            issue(1, slots[1], h)

            def pair(i, _, h=h):
                process(i * 2, slots[0], h)
                process(i * 2 + 1, slots[1], h)
                return 0

            lax.fori_loop(0, NGRP // 2, pair, 0)
            plsc.subcore_barrier()
            pltpu.sync_copy(acc.at[pl.ds(s * ROWS_T, ROWS_T)],
                            outp.at[c, h, pl.ds(s * ROWS_T, ROWS_T)])
            plsc.subcore_barrier()
            return 0

        lax.fori_loop(0, H, head, 0)

    return pl.kernel(
        body,
        out_type=(jax.ShapeDtypeStruct((2, H, NP, HID), _f32),),
        mesh=mesh,
        scratch_types=[
            pltpu.VMEM((NGS, GW), _i32),
            pltpu.VMEM((NGS, GW), _i32),
            pltpu.VMEM((GW,), _i32),
            pltpu.VMEM((GW,), _i32),
            pltpu.VMEM((GW,), _f32),
            pltpu.VMEM((GW,), _f32),
            pltpu.VMEM((GW, HID), _f32),
            pltpu.VMEM((GW, HID), _f32),
            pltpu.VMEM_SHARED((NP, HID), _f32),
            pltpu.SemaphoreType.DMA,
            pltpu.SemaphoreType.DMA,
            pltpu.SemaphoreType.DMA,
            pltpu.SemaphoreType.DMA,
        ],
    )


def _mm2(x, w1, w2):
    npad, k = x.shape
    f = w1.shape[1]
    br = 128
    grid = (npad // br,)

    def body(x_ref, w1_ref, w2_ref, o1_ref, o2_ref):
        xb = x_ref[...]
        o1_ref[...] = jnp.dot(xb, w1_ref[...], preferred_element_type=_f32)
        o2_ref[...] = jnp.dot(xb, w2_ref[...], preferred_element_type=_f32)

    return pl.pallas_call(
        body,
        grid=grid,
        in_specs=[
            pl.BlockSpec((br, k), lambda i: (i, 0)),
            pl.BlockSpec((k, f), lambda i: (0, 0)),
            pl.BlockSpec((k, f), lambda i: (0, 0)),
        ],
        out_specs=[pl.BlockSpec((br, f), lambda i: (i, 0))] * 2,
        out_shape=[jax.ShapeDtypeStruct((npad, f), _f32)] * 2,
    )(x, w1, w2)


def _combine_stats(outp, denp, H):
    f = H * HID
    br = 128
    grid = (NP // br,)

    def body(op_ref, dp_ref, y_ref, st_ref, acc_ref):
        g = pl.program_id(0)
        num = op_ref[0] + op_ref[1]
        den = dp_ref[0] + dp_ref[1]
        yb = num.transpose(1, 0, 2) / (den.T[:, :, None] + 1e-16)
        yb = yb.reshape(br, f)
        y_ref[...] = yb

        @pl.when(g == 0)
        def _():
            acc_ref[...] = jnp.zeros_like(acc_ref)

        acc_ref[0, :] += jnp.sum(yb, axis=0)
        acc_ref[1, :] += jnp.sum(yb * yb, axis=0)

        @pl.when(g == NP // br - 1)
        def _():
            st_ref[...] = acc_ref[...]

    return pl.pallas_call(
        body,
        grid=grid,
        in_specs=[
            pl.BlockSpec((2, H, br, HID), lambda i: (0, 0, i, 0)),
            pl.BlockSpec((2, H, br), lambda i: (0, 0, i)),
        ],
        out_specs=[
            pl.BlockSpec((br, f), lambda i: (i, 0)),
            pl.BlockSpec((8, f), lambda i: (0, 0)),
        ],
        out_shape=[
            jax.ShapeDtypeStruct((NP, f), _f32),
            jax.ShapeDtypeStruct((8, f), _f32),
        ],
        scratch_shapes=[pltpu.VMEM((8, f), _f32)],
    )(outp, denp)


def _bn_block(y, st, g, b):
    mu = st[0, :] * (1.0 / N)
    var = st[1, :] * (1.0 / N) - mu * mu
    inv = lax.rsqrt(var + 1e-5)
    return (y - mu[None, :]) * (inv * g)[None, :] + b[None, :]


def _norm_proj_mm(y, st, bn_g, bn_b, pw, pb, wl, wr):
    f = y.shape[1]
    br = 128
    grid = (NP // br,)

    def body(y_ref, st_ref, g_ref, b_ref, pw_ref, pb_ref, wl_ref, wr_ref,
             xp_ref, xl_ref, xr_ref):
        x1 = jnp.maximum(
            _bn_block(y_ref[...], st_ref[...], g_ref[0], b_ref[0]), 0.0)
        xp = jnp.dot(x1, pw_ref[...], preferred_element_type=_f32) + pb_ref[0]
        xp_ref[...] = xp
        xl_ref[...] = jnp.dot(xp, wl_ref[...], preferred_element_type=_f32)
        xr_ref[...] = jnp.dot(xp, wr_ref[...], preferred_element_type=_f32)

    return pl.pallas_call(
        body,
        grid=grid,
        in_specs=[
            pl.BlockSpec((br, f), lambda i: (i, 0)),
            pl.BlockSpec((8, f), lambda i: (0, 0)),
            pl.BlockSpec((1, f), lambda i: (0, 0)),
            pl.BlockSpec((1, f), lambda i: (0, 0)),
            pl.BlockSpec((f, HID), lambda i: (0, 0)),
            pl.BlockSpec((1, HID), lambda i: (0, 0)),
            pl.BlockSpec((HID, HID), lambda i: (0, 0)),
            pl.BlockSpec((HID, HID), lambda i: (0, 0)),
        ],
        out_specs=[pl.BlockSpec((br, HID), lambda i: (i, 0))] * 3,
        out_shape=[jax.ShapeDtypeStruct((NP, HID), _f32)] * 3,
    )(y, st, bn_g, bn_b, pw, pb, wl, wr)


def _norm_res_mm(y, st, bn_g, bn_b, res, wl, wr):
    br = 128
    grid = (NP // br,)

    def body(y_ref, st_ref, g_ref, b_ref, r_ref, wl_ref, wr_ref,
             xt_ref, xl_ref, xr_ref):
        xt = jnp.maximum(
            _bn_block(y_ref[...], st_ref[...], g_ref[0], b_ref[0]), 0.0)
        xt = xt + r_ref[...]
        xt_ref[...] = xt
        xl_ref[...] = jnp.dot(xt, wl_ref[...], preferred_element_type=_f32)
        xr_ref[...] = jnp.dot(xt, wr_ref[...], preferred_element_type=_f32)

    return pl.pallas_call(
        body,
        grid=grid,
        in_specs=[
            pl.BlockSpec((br, HID), lambda i: (i, 0)),
            pl.BlockSpec((8, HID), lambda i: (0, 0)),
            pl.BlockSpec((1, HID), lambda i: (0, 0)),
            pl.BlockSpec((1, HID), lambda i: (0, 0)),
            pl.BlockSpec((br, HID), lambda i: (i, 0)),
            pl.BlockSpec((HID, HID), lambda i: (0, 0)),
            pl.BlockSpec((HID, HID), lambda i: (0, 0)),
        ],
        out_specs=[pl.BlockSpec((br, HID), lambda i: (i, 0))] * 3,
        out_shape=[jax.ShapeDtypeStruct((NP, HID), _f32)] * 3,
    )(y, st, bn_g, bn_b, res, wl, wr)


def _norm_res_pool(y, st, bn_g, bn_b, res, batch3d, lw, lb):
    br = 128
    grid = (NP // br,)

    def body(y_ref, st_ref, g_ref, b_ref, r_ref, bt_ref, lw_ref, lb_ref,
             o_ref, acc_ref):
        g = pl.program_id(0)
        x5 = jnp.maximum(
            _bn_block(y_ref[...], st_ref[...], g_ref[0], b_ref[0]), 0.0)
        x5 = x5 + r_ref[...]
        bt = bt_ref[0]
        oh = jnp.where(
            bt.reshape(br, 1) == lax.broadcasted_iota(_i32, (br, NG), 1),
            1.0, 0.0)

        @pl.when(g == 0)
        def _():
            acc_ref[...] = jnp.zeros_like(acc_ref)

        acc_ref[...] += lax.dot_general(oh, x5, (((0,), (0,)), ((), ())),
                                        preferred_element_type=_f32)

        @pl.when(g == NP // br - 1)
        def _():
            o_ref[...] = jnp.dot(acc_ref[...], lw_ref[...],
                                 preferred_element_type=_f32) + lb_ref[0]

    return pl.pallas_call(
        body,
        grid=grid,
        in_specs=[
            pl.BlockSpec((br, HID), lambda i: (i, 0)),
            pl.BlockSpec((8, HID), lambda i: (0, 0)),
            pl.BlockSpec((1, HID), lambda i: (0, 0)),
            pl.BlockSpec((1, HID), lambda i: (0, 0)),
            pl.BlockSpec((br, HID), lambda i: (i, 0)),
            pl.BlockSpec((1, 1, br), lambda i: (i, 0, 0)),
            pl.BlockSpec((HID, 128), lambda i: (0, 0)),
            pl.BlockSpec((1, 128), lambda i: (0, 0)),
        ],
        out_specs=[pl.BlockSpec((NG, 128), lambda i: (0, 0))],
        out_shape=[jax.ShapeDtypeStruct((NG, 128), _f32)],
        scratch_shapes=[pltpu.VMEM((NG, HID), _f32)],
    )(y, st, bn_g, bn_b, res, batch3d, lw, lb)


def _gat_layer(xl, xr, src2d, dst2d, att, H):
    xlh = xl.reshape(NP * H, HID)
    xrh = xr.reshape(NP * H, HID)
    ex, denp = _sc_scores(H)(xlh, xrh, src2d, dst2d, att)
    (outp,) = _sc_aggregate(H)(xlh, src2d, dst2d, ex)
    denp = denp.reshape(2, H, NPD * HID)[:, :, :NP]
    return _combine_stats(outp, denp, H)


def kernel(x, edge_index, batch, gat1_Wl, gat1_Wr, gat1_att, gat1_b,
           proj1_W, proj1_b, gat2_Wl, gat2_Wr, gat2_att, gat2_b,
           gat3_Wl, gat3_Wr, gat3_att, gat3_b, gat4_Wl, gat4_Wr, gat4_att,
           gat4_b, gat5_Wl, gat5_Wr, gat5_att, gat5_b, bn1_g, bn1_b,
           bn2_g, bn2_b, bn3_g, bn3_b, bn4_g, bn4_b, bn5_g, bn5_b,
           lin_W, lin_b):
    loop = jnp.arange(N, dtype=jnp.int32)
    src = jnp.concatenate([edge_index[0], loop, jnp.zeros((1,), jnp.int32)])
    dst = jnp.concatenate([edge_index[1], loop, jnp.zeros((1,), jnp.int32)])
    src_p = src[_EDGE_MAP]
    dst_p = dst[_EDGE_MAP]
    src2d = src_p.reshape(E2P // GW, GW)
    dst2d = dst_p.reshape(E2P // GW, GW)
    xp0 = jnp.pad(x, ((0, NP - N), (0, 0)))
    batch3d = jnp.pad(batch, (0, NP - N), constant_values=NG).reshape(
        NP // GW, 1, GW)
    row = lambda v: v.reshape(1, -1)
    lwp = jnp.pad(lin_W, ((0, 0), (0, 128 - NCLS)))
    lbp = jnp.pad(lin_b, (0, 128 - NCLS)).reshape(1, 128)

    xl1, xr1 = _mm2(xp0, gat1_Wl, gat1_Wr)
    y1, st1 = _gat_layer(xl1, xr1, src2d, dst2d, gat1_att, HEADS)
    x1p, xl2, xr2 = _norm_proj_mm(y1, st1, row(bn1_g), row(bn1_b),
                                  proj1_W, row(proj1_b), gat2_Wl, gat2_Wr)

    y2, st2 = _gat_layer(xl2, xr2, src2d, dst2d, gat2_att, 1)
    x2, xl3, xr3 = _norm_res_mm(y2, st2, row(bn2_g), row(bn2_b), x1p,
                                gat3_Wl, gat3_Wr)
    y3, st3 = _gat_layer(xl3, xr3, src2d, dst2d, gat3_att, 1)
    x3, xl4, xr4 = _norm_res_mm(y3, st3, row(bn3_g), row(bn3_b), x2,
                                gat4_Wl, gat4_Wr)
    y4, st4 = _gat_layer(xl4, xr4, src2d, dst2d, gat4_att, 1)
    x4, xl5, xr5 = _norm_res_mm(y4, st4, row(bn4_g), row(bn4_b), x3,
                                gat5_Wl, gat5_Wr)

    y5, st5 = _gat_layer(xl5, xr5, src2d, dst2d, gat5_att, 1)
    (outp,) = _norm_res_pool(y5, st5, row(bn5_g), row(bn5_b), x4,
                             batch3d, lwp, lbp)
    return outp[:, :NCLS]


# speedup vs baseline: 9.8201x; 1.0020x over previous
import functools

import jax
import jax.numpy as jnp
from jax import lax
from jax.experimental import pallas as pl
from jax.experimental.pallas import tpu as pltpu
from jax.experimental.pallas import tpu_sc as plsc

N = 10000
E = 160000
E2 = E + N
DIN = 128
HID = 128
HEADS = 4
NCLS = 7
NG = 64

NP = 10112
GW = 128
NW = 32
NGRP = 42
NGS = 48
CHS = NGS * GW
E2P = NW * CHS
CNT_LO = E2 // NW
CNT_HI = CNT_LO + 1
ROWS_T = NP // 16
NPD = 80

_f32 = jnp.float32
_i32 = jnp.int32


def _edge_map():
    import numpy as np
    m = np.full((E2P,), E2, np.int32)
    pos = 0
    for w in range(NW):
        c = CNT_HI if w < 16 else CNT_LO
        m[w * CHS:w * CHS + c] = np.arange(pos, pos + c, dtype=np.int32)
        pos += c
    assert pos == E2
    return m


_EDGE_MAP = _edge_map()


@functools.lru_cache(maxsize=None)
def _sc_scores(H):
    mesh = plsc.VectorSubcoreMesh(core_axis_name="c", subcore_axis_name="s")

    def body(xlh, xrh, src2d, dst2d, att_h, ex_h, denp,
             src_v, dst_v, idxs0, idxs1, idxd0, idxd1, xl0, xl1, xr0, xr1,
             ex0, ex1, sd0, sd1, id0, id1, att_v, acc_den,
             sl0, sl1, sr0, sr1, sdm0, sdm1, sx0, sx1):
        c = lax.axis_index("c")
        s = lax.axis_index("s")
        wid = s * 2 + c
        cnt = jnp.where(wid < 16, CNT_HI, CNT_LO)
        pltpu.sync_copy(src2d.at[pl.ds(wid * NGS, NGS)], src_v)
        pltpu.sync_copy(dst2d.at[pl.ds(wid * NGS, NGS)], dst_v)
        pltpu.sync_copy(att_h, att_v)
        iota = lax.iota(_i32, 16)
        zv = jnp.zeros((16,), _f32)
        slots = ((idxs0, idxd0, xl0, xr0, sl0, sr0, sd0, id0, sdm0, ex0,
                  sx0),
                 (idxs1, idxd1, xl1, xr1, sl1, sr1, sd1, id1, sdm1, ex1,
                  sx1))

        def zrow(i, _):
            for v in range(HID // 16):
                sd0[i, pl.ds(v * 16, 16)] = zv
            return 0

        def issue(g, slot, h):
            idxs, idxd, xl_b, xr_b, s_l, s_r = slot[:6]
            for j in range(8):
                sv = src_v[g, pl.ds(j * 16, 16)]
                idxs[pl.ds(j * 16, 16)] = sv * H + h
                dv = dst_v[g, pl.ds(j * 16, 16)]
                idxd[pl.ds(j * 16, 16)] = dv * H + h
            pltpu.async_copy(xlh.at[idxs], xl_b, s_l)
            pltpu.async_copy(xrh.at[idxd], xr_b, s_r)

        def process(g, slot, h, att_k):
            (idxs, idxd, xl_b, xr_b, s_l, s_r, sd_b, id_b, s_d, ex_b,
             s_x) = slot
            pltpu.make_async_copy(xlh.at[idxs], xl_b, s_l).wait()
            pltpu.make_async_copy(xrh.at[idxd], xr_b, s_r).wait()
            for sg in range(8):

                def edge(i, sv, sg=sg):
                    e = sg * 16 + i
                    acc = jnp.zeros((16,), _f32)
                    for k in range(HID // 16):
                        m = (xl_b[e, pl.ds(k * 16, 16)]
                             + xr_b[e, pl.ds(k * 16, 16)])
                        m = jnp.maximum(m, m * 0.2)
                        acc = acc + m * att_k[k]
                    for b in (1, 2, 4, 8):
                        perm = jnp.bitwise_xor(iota, b)
                        acc = acc + acc.at[perm].get(
                            mode="promise_in_bounds")
                    return jnp.where(iota == i, acc, sv)

                score = lax.fori_loop(0, 16, edge, jnp.zeros((16,), _f32))
                le = iota + (g * GW + sg * 16)
                exv = jnp.where(le < cnt, jnp.exp(score), 0.0)

                if sg == 0:
                    @pl.when(g >= 2)
                    def _():
                        pltpu.make_async_copy(
                            ex_b,
                            ex_h.at[pl.ds(
                                h * E2P + wid * CHS + (g - 2) * GW, GW)],
                            s_x).wait()

                ex_b[pl.ds(sg * 16, 16)] = exv
            pltpu.async_copy(
                ex_b, ex_h.at[pl.ds(h * E2P + wid * CHS + g * GW, GW)], s_x)
            @pl.when(g < NGRP - 2)
            def _():
                issue(g + 2, slot, h)
            @pl.when(g >= 2)
            def _():
                pltpu.make_async_copy(sd_b, acc_den.at[id_b], s_d).wait()

            for j in range(8):
                dv = dst_v[g, pl.ds(j * 16, 16)]
                id_b[pl.ds(j * 16, 16)] = jnp.right_shift(dv, 7)
            for sg in range(8):
                exv = ex_b[pl.ds(sg * 16, 16)]
                dmod = jnp.bitwise_and(dst_v[g, pl.ds(sg * 16, 16)], 127)

                def row(i, _, exv=exv, dmod=dmod, sg=sg):
                    e = sg * 16 + i
                    lane = jnp.full((16,), i, _i32)
                    sc = exv.at[lane].get(mode="promise_in_bounds")
                    dm = dmod.at[lane].get(mode="promise_in_bounds")
                    for v in range(HID // 16):
                        sd_b[e, pl.ds(v * 16, 16)] = jnp.where(
                            iota + (v * 16) == dm, sc, 0.0)
                    return 0

                lax.fori_loop(0, 16, row, 0)
            pltpu.async_copy(sd_b, acc_den.at[id_b], s_d, add=True)

        def head(h, _):
            att_k = [att_v[h, pl.ds(k * 16, 16)] for k in range(HID // 16)]

            @pl.when(s == 0)
            def _():
                lax.fori_loop(0, NPD, zrow, 0)
                pltpu.sync_copy(sd0.at[pl.ds(0, NPD)], acc_den)

            plsc.subcore_barrier()
            issue(0, slots[0], h)
            issue(1, slots[1], h)

            def pair(i, _, h=h, att_k=att_k):
                process(i * 2, slots[0], h, att_k)
                process(i * 2 + 1, slots[1], h, att_k)
                return 0

            lax.fori_loop(0, NGRP // 2, pair, 0)
            pltpu.make_async_copy(sd0, acc_den.at[id0], sdm0).wait()
            pltpu.make_async_copy(sd1, acc_den.at[id1], sdm1).wait()
            pltpu.make_async_copy(
                ex0, ex_h.at[pl.ds(h * E2P + wid * CHS + (NGRP - 2) * GW,
                                   GW)], sx0).wait()
            pltpu.make_async_copy(
                ex1, ex_h.at[pl.ds(h * E2P + wid * CHS + (NGRP - 1) * GW,
                                   GW)], sx1).wait()
            plsc.subcore_barrier()

            @pl.when(s == 0)
            def _():
                pltpu.sync_copy(acc_den, denp.at[c, h])

            plsc.subcore_barrier()
            return 0

        lax.fori_loop(0, H, head, 0)

    return pl.kernel(
        body,
        out_type=(jax.ShapeDtypeStruct((H * E2P,), _f32),
                  jax.ShapeDtypeStruct((2, H, NPD, HID), _f32)),
        mesh=mesh,
        scratch_types=[
            pltpu.VMEM((NGS, GW), _i32),
            pltpu.VMEM((NGS, GW), _i32),
            pltpu.VMEM((GW,), _i32),
            pltpu.VMEM((GW,), _i32),
            pltpu.VMEM((GW,), _i32),
            pltpu.VMEM((GW,), _i32),
            pltpu.VMEM((GW, HID), _f32),
            pltpu.VMEM((GW, HID), _f32),
            pltpu.VMEM((GW, HID), _f32),
            pltpu.VMEM((GW, HID), _f32),
            pltpu.VMEM((GW,), _f32),
            pltpu.VMEM((GW,), _f32),
            pltpu.VMEM((GW, HID), _f32),
            pltpu.VMEM((GW, HID), _f32),
            pltpu.VMEM((GW,), _i32),
            pltpu.VMEM((GW,), _i32),
            pltpu.VMEM((H, HID), _f32),
            pltpu.VMEM_SHARED((NPD, HID), _f32),
            pltpu.SemaphoreType.DMA,
            pltpu.SemaphoreType.DMA,
            pltpu.SemaphoreType.DMA,
            pltpu.SemaphoreType.DMA,
            pltpu.SemaphoreType.DMA,
            pltpu.SemaphoreType.DMA,
            pltpu.SemaphoreType.DMA,
            pltpu.SemaphoreType.DMA,
        ],
    )


@functools.lru_cache(maxsize=None)
def _sc_aggregate(H):
    mesh = plsc.VectorSubcoreMesh(core_axis_name="c", subcore_axis_name="s")

    def body(xlh, src2d, dst2d, ex_h, outp,
             src_v, dst_v, dha, dhb, idxs0, idxs1, ex0, ex1, xl0, xl1,
             acc, sl0, sl1, se0, se1, sa, sb):
        c = lax.axis_index("c")
        s = lax.axis_index("s")
        wid = s * 2 + c
        zv = jnp.zeros((16,), _f32)
        pltpu.sync_copy(src2d.at[pl.ds(wid * NGS, NGS)], src_v)
        pltpu.sync_copy(dst2d.at[pl.ds(wid * NGS, NGS)], dst_v)
        slots = ((idxs0, ex0, xl0, sl0, se0),
                 (idxs1, ex1, xl1, sl1, se1))

        def zrow(i, _):
            for v in range(HID // 16):
                xl0[i, pl.ds(v * 16, 16)] = zv
            return 0

        def issue(g, slot, h):
            idxs, ex_b, xl_b, s_l, s_e = slot
            for j in range(8):
                sv = src_v[g, pl.ds(j * 16, 16)]
                idxs[pl.ds(j * 16, 16)] = sv * H + h
            pltpu.async_copy(xlh.at[idxs], xl_b, s_l)
            pltpu.async_copy(
                ex_h.at[pl.ds(h * E2P + wid * CHS + g * GW, GW)], ex_b, s_e)

        def process(g, slot, h):
            idxs, ex_b, xl_b, s_l, s_e = slot
            pltpu.make_async_copy(xlh.at[idxs], xl_b, s_l).wait()
            pltpu.make_async_copy(
                ex_h.at[pl.ds(h * E2P + wid * CHS + g * GW, GW)],
                ex_b, s_e).wait()
            def half(lo):
                for sg in range(lo, lo + 4):
                    exv = ex_b[pl.ds(sg * 16, 16)]

                    def row(i, _, exv=exv, sg=sg):
                        e = sg * 16 + i
                        lane = jnp.full((16,), i, _i32)
                        sc = exv.at[lane].get(mode="promise_in_bounds")
                        for v in range(HID // 16):
                            xl_b[e, pl.ds(v * 16, 16)] = (
                                xl_b[e, pl.ds(v * 16, 16)] * sc)
                        return 0

                    lax.fori_loop(0, 16, row, 0)

            for j in range(4):
                dha[pl.ds(j * 16, 16)] = dst_v[g, pl.ds(j * 16, 16)]
                dhb[pl.ds(j * 16, 16)] = dst_v[g, pl.ds(64 + j * 16, 16)]
            half(0)
            pltpu.async_copy(xl_b.at[pl.ds(0, 64)],
                             acc.at[dha], sa, add=True)
            half(4)
            pltpu.async_copy(xl_b.at[pl.ds(64, 64)],
                             acc.at[dhb], sb, add=True)
            pltpu.make_async_copy(xl_b.at[pl.ds(0, 64)],
                                  acc.at[dha], sa).wait()
            pltpu.make_async_copy(xl_b.at[pl.ds(64, 64)],
                                  acc.at[dhb], sb).wait()

            @pl.when(g < NGRP - 2)
            def _():
                issue(g + 2, slot, h)

        def head(h, _):
            lax.fori_loop(0, GW, zrow, 0)
            for zoff, zlen in ((0, 128), (128, 128), (256, 128),
                               (384, 128), (512, 120)):
                pltpu.sync_copy(
                    xl0.at[pl.ds(0, zlen)],
                    acc.at[pl.ds(s * ROWS_T + zoff, zlen)])
            plsc.subcore_barrier()
            issue(0, slots[0], h)
            issue(1, slots[1], h)

            def pair(i, _, h=h):
                process(i * 2, slots[0], h)
                process(i * 2 + 1, slots[1], h)
                return 0

            lax.fori_loop(0, NGRP // 2, pair, 0)
            plsc.subcore_barrier()
            pltpu.sync_copy(acc.at[pl.ds(s * ROWS_T, ROWS_T)],
                            outp.at[c, h, pl.ds(s * ROWS_T, ROWS_T)])
            plsc.subcore_barrier()
            return 0

        lax.fori_loop(0, H, head, 0)

    return pl.kernel(
        body,
        out_type=(jax.ShapeDtypeStruct((2, H, NP, HID), _f32),),
        mesh=mesh,
        scratch_types=[
            pltpu.VMEM((NGS, GW), _i32),
            pltpu.VMEM((NGS, GW), _i32),
            pltpu.VMEM((64,), _i32),
            pltpu.VMEM((64,), _i32),
            pltpu.VMEM((GW,), _i32),
            pltpu.VMEM((GW,), _i32),
            pltpu.VMEM((GW,), _f32),
            pltpu.VMEM((GW,), _f32),
            pltpu.VMEM((GW, HID), _f32),
            pltpu.VMEM((GW, HID), _f32),
            pltpu.VMEM_SHARED((NP, HID), _f32),
            pltpu.SemaphoreType.DMA,
            pltpu.SemaphoreType.DMA,
            pltpu.SemaphoreType.DMA,
            pltpu.SemaphoreType.DMA,
            pltpu.SemaphoreType.DMA,
            pltpu.SemaphoreType.DMA,
        ],
    )


def _mm2(x, w1, w2):
    npad, k = x.shape
    f = w1.shape[1]
    br = 128
    grid = (npad // br,)

    def body(x_ref, w1_ref, w2_ref, o1_ref, o2_ref):
        xb = x_ref[...]
        o1_ref[...] = jnp.dot(xb, w1_ref[...], preferred_element_type=_f32)
        o2_ref[...] = jnp.dot(xb, w2_ref[...], preferred_element_type=_f32)

    return pl.pallas_call(
        body,
        grid=grid,
        in_specs=[
            pl.BlockSpec((br, k), lambda i: (i, 0)),
            pl.BlockSpec((k, f), lambda i: (0, 0)),
            pl.BlockSpec((k, f), lambda i: (0, 0)),
        ],
        out_specs=[pl.BlockSpec((br, f), lambda i: (i, 0))] * 2,
        out_shape=[jax.ShapeDtypeStruct((npad, f), _f32)] * 2,
    )(x, w1, w2)


def _combine_stats(outp, denp, H):
    f = H * HID
    br = 128
    grid = (NP // br,)

    def body(op_ref, dp_ref, y_ref, st_ref, acc_ref):
        g = pl.program_id(0)
        num = op_ref[0] + op_ref[1]
        den = dp_ref[0] + dp_ref[1]
        yb = num.transpose(1, 0, 2) / (den.T[:, :, None] + 1e-16)
        yb = yb.reshape(br, f)
        y_ref[...] = yb

        @pl.when(g == 0)
        def _():
            acc_ref[...] = jnp.zeros_like(acc_ref)

        acc_ref[0, :] += jnp.sum(yb, axis=0)
        acc_ref[1, :] += jnp.sum(yb * yb, axis=0)

        @pl.when(g == NP // br - 1)
        def _():
            st_ref[...] = acc_ref[...]

    return pl.pallas_call(
        body,
        grid=grid,
        in_specs=[
            pl.BlockSpec((2, H, br, HID), lambda i: (0, 0, i, 0)),
            pl.BlockSpec((2, H, br), lambda i: (0, 0, i)),
        ],
        out_specs=[
            pl.BlockSpec((br, f), lambda i: (i, 0)),
            pl.BlockSpec((8, f), lambda i: (0, 0)),
        ],
        out_shape=[
            jax.ShapeDtypeStruct((NP, f), _f32),
            jax.ShapeDtypeStruct((8, f), _f32),
        ],
        scratch_shapes=[pltpu.VMEM((8, f), _f32)],
    )(outp, denp)


def _bn_block(y, st, g, b):
    mu = st[0, :] * (1.0 / N)
    var = st[1, :] * (1.0 / N) - mu * mu
    inv = lax.rsqrt(var + 1e-5)
    return (y - mu[None, :]) * (inv * g)[None, :] + b[None, :]


def _norm_proj_mm(y, st, bn_g, bn_b, pw, pb, wl, wr):
    f = y.shape[1]
    br = 128
    grid = (NP // br,)

    def body(y_ref, st_ref, g_ref, b_ref, pw_ref, pb_ref, wl_ref, wr_ref,
             xp_ref, xl_ref, xr_ref):
        x1 = jnp.maximum(
            _bn_block(y_ref[...], st_ref[...], g_ref[0], b_ref[0]), 0.0)
        xp = jnp.dot(x1, pw_ref[...], preferred_element_type=_f32) + pb_ref[0]
        xp_ref[...] = xp
        xl_ref[...] = jnp.dot(xp, wl_ref[...], preferred_element_type=_f32)
        xr_ref[...] = jnp.dot(xp, wr_ref[...], preferred_element_type=_f32)

    return pl.pallas_call(
        body,
        grid=grid,
        in_specs=[
            pl.BlockSpec((br, f), lambda i: (i, 0)),
            pl.BlockSpec((8, f), lambda i: (0, 0)),
            pl.BlockSpec((1, f), lambda i: (0, 0)),
            pl.BlockSpec((1, f), lambda i: (0, 0)),
            pl.BlockSpec((f, HID), lambda i: (0, 0)),
            pl.BlockSpec((1, HID), lambda i: (0, 0)),
            pl.BlockSpec((HID, HID), lambda i: (0, 0)),
            pl.BlockSpec((HID, HID), lambda i: (0, 0)),
        ],
        out_specs=[pl.BlockSpec((br, HID), lambda i: (i, 0))] * 3,
        out_shape=[jax.ShapeDtypeStruct((NP, HID), _f32)] * 3,
    )(y, st, bn_g, bn_b, pw, pb, wl, wr)


def _norm_res_mm(y, st, bn_g, bn_b, res, wl, wr):
    br = 128
    grid = (NP // br,)

    def body(y_ref, st_ref, g_ref, b_ref, r_ref, wl_ref, wr_ref,
             xt_ref, xl_ref, xr_ref):
        xt = jnp.maximum(
            _bn_block(y_ref[...], st_ref[...], g_ref[0], b_ref[0]), 0.0)
        xt = xt + r_ref[...]
        xt_ref[...] = xt
        xl_ref[...] = jnp.dot(xt, wl_ref[...], preferred_element_type=_f32)
        xr_ref[...] = jnp.dot(xt, wr_ref[...], preferred_element_type=_f32)

    return pl.pallas_call(
        body,
        grid=grid,
        in_specs=[
            pl.BlockSpec((br, HID), lambda i: (i, 0)),
            pl.BlockSpec((8, HID), lambda i: (0, 0)),
            pl.BlockSpec((1, HID), lambda i: (0, 0)),
            pl.BlockSpec((1, HID), lambda i: (0, 0)),
            pl.BlockSpec((br, HID), lambda i: (i, 0)),
            pl.BlockSpec((HID, HID), lambda i: (0, 0)),
            pl.BlockSpec((HID, HID), lambda i: (0, 0)),
        ],
        out_specs=[pl.BlockSpec((br, HID), lambda i: (i, 0))] * 3,
        out_shape=[jax.ShapeDtypeStruct((NP, HID), _f32)] * 3,
    )(y, st, bn_g, bn_b, res, wl, wr)


def _norm_res_pool(y, st, bn_g, bn_b, res, batch3d, lw, lb):
    br = 128
    grid = (NP // br,)

    def body(y_ref, st_ref, g_ref, b_ref, r_ref, bt_ref, lw_ref, lb_ref,
             o_ref, acc_ref):
        g = pl.program_id(0)
        x5 = jnp.maximum(
            _bn_block(y_ref[...], st_ref[...], g_ref[0], b_ref[0]), 0.0)
        x5 = x5 + r_ref[...]
        bt = bt_ref[0]
        oh = jnp.where(
            bt.reshape(br, 1) == lax.broadcasted_iota(_i32, (br, NG), 1),
            1.0, 0.0)

        @pl.when(g == 0)
        def _():
            acc_ref[...] = jnp.zeros_like(acc_ref)

        acc_ref[...] += lax.dot_general(oh, x5, (((0,), (0,)), ((), ())),
                                        preferred_element_type=_f32)

        @pl.when(g == NP // br - 1)
        def _():
            o_ref[...] = jnp.dot(acc_ref[...], lw_ref[...],
                                 preferred_element_type=_f32) + lb_ref[0]

    return pl.pallas_call(
        body,
        grid=grid,
        in_specs=[
            pl.BlockSpec((br, HID), lambda i: (i, 0)),
            pl.BlockSpec((8, HID), lambda i: (0, 0)),
            pl.BlockSpec((1, HID), lambda i: (0, 0)),
            pl.BlockSpec((1, HID), lambda i: (0, 0)),
            pl.BlockSpec((br, HID), lambda i: (i, 0)),
            pl.BlockSpec((1, 1, br), lambda i: (i, 0, 0)),
            pl.BlockSpec((HID, 128), lambda i: (0, 0)),
            pl.BlockSpec((1, 128), lambda i: (0, 0)),
        ],
        out_specs=[pl.BlockSpec((NG, 128), lambda i: (0, 0))],
        out_shape=[jax.ShapeDtypeStruct((NG, 128), _f32)],
        scratch_shapes=[pltpu.VMEM((NG, HID), _f32)],
    )(y, st, bn_g, bn_b, res, batch3d, lw, lb)


def _gat_layer(xl, xr, src2d, dst2d, dst2d64, att, H):
    xlh = xl.reshape(NP * H, HID)
    xrh = xr.reshape(NP * H, HID)
    ex, denp = _sc_scores(H)(xlh, xrh, src2d, dst2d, att)
    (outp,) = _sc_aggregate(H)(xlh, src2d, dst2d, ex)
    denp = denp.reshape(2, H, NPD * HID)[:, :, :NP]
    return _combine_stats(outp, denp, H)


def kernel(x, edge_index, batch, gat1_Wl, gat1_Wr, gat1_att, gat1_b,
           proj1_W, proj1_b, gat2_Wl, gat2_Wr, gat2_att, gat2_b,
           gat3_Wl, gat3_Wr, gat3_att, gat3_b, gat4_Wl, gat4_Wr, gat4_att,
           gat4_b, gat5_Wl, gat5_Wr, gat5_att, gat5_b, bn1_g, bn1_b,
           bn2_g, bn2_b, bn3_g, bn3_b, bn4_g, bn4_b, bn5_g, bn5_b,
           lin_W, lin_b):
    loop = jnp.arange(N, dtype=jnp.int32)
    src = jnp.concatenate([edge_index[0], loop, jnp.zeros((1,), jnp.int32)])
    dst = jnp.concatenate([edge_index[1], loop, jnp.zeros((1,), jnp.int32)])
    src_p = src[_EDGE_MAP]
    dst_p = dst[_EDGE_MAP]
    src2d = src_p.reshape(E2P // GW, GW)
    dst2d = dst_p.reshape(E2P // GW, GW)
    dst2d64 = dst_p.reshape(E2P // 64, 64)
    xp0 = jnp.pad(x, ((0, NP - N), (0, 0)))
    batch3d = jnp.pad(batch, (0, NP - N), constant_values=NG).reshape(
        NP // GW, 1, GW)
    row = lambda v: v.reshape(1, -1)
    lwp = jnp.pad(lin_W, ((0, 0), (0, 128 - NCLS)))
    lbp = jnp.pad(lin_b, (0, 128 - NCLS)).reshape(1, 128)

    xl1, xr1 = _mm2(xp0, gat1_Wl, gat1_Wr)
    y1, st1 = _gat_layer(xl1, xr1, src2d, dst2d, dst2d64, gat1_att, HEADS)
    x1p, xl2, xr2 = _norm_proj_mm(y1, st1, row(bn1_g), row(bn1_b),
                                  proj1_W, row(proj1_b), gat2_Wl, gat2_Wr)

    y2, st2 = _gat_layer(xl2, xr2, src2d, dst2d, dst2d64, gat2_att, 1)
    x2, xl3, xr3 = _norm_res_mm(y2, st2, row(bn2_g), row(bn2_b), x1p,
                                gat3_Wl, gat3_Wr)
    y3, st3 = _gat_layer(xl3, xr3, src2d, dst2d, dst2d64, gat3_att, 1)
    x3, xl4, xr4 = _norm_res_mm(y3, st3, row(bn3_g), row(bn3_b), x2,
                                gat4_Wl, gat4_Wr)
    y4, st4 = _gat_layer(xl4, xr4, src2d, dst2d, dst2d64, gat4_att, 1)
    x4, xl5, xr5 = _norm_res_mm(y4, st4, row(bn4_g), row(bn4_b), x3,
                                gat5_Wl, gat5_Wr)

    y5, st5 = _gat_layer(xl5, xr5, src2d, dst2d, dst2d64, gat5_att, 1)
    (outp,) = _norm_res_pool(y5, st5, row(bn5_g), row(bn5_b), x4,
                             batch3d, lwp, lbp)
    return outp[:, :NCLS]
